# baseline (speedup 1.0000x reference)
import itertools

import jax
import jax.numpy as jnp
import numpy as np
from jax.experimental import pallas as pl

_NUM_GPTS = 128
_HYPS = 64
_INLIER_SIGMA2 = 0.01
_INLIER_ALPHA = 0.5

_SEL_CACHE = {}


def _get_sel(B):
    if B not in _SEL_CACHE:
        idx_combi = np.array(
            list(itertools.combinations(range(_NUM_GPTS), 3)), dtype=np.int32)
        with jax.ensure_compile_time_eval():
            tmp = np.asarray(
                jax.random.randint(jax.random.key(42), (_HYPS * B,), 0,
                                   idx_combi.shape[0]))
        combos = idx_combi[tmp].reshape(B, _HYPS, 3)
        _SEL_CACHE[B] = (np.ascontiguousarray(combos[:, :, 0]),
                         np.ascontiguousarray(combos[:, :, 1]),
                         np.ascontiguousarray(combos[:, :, 2]))
    return _SEL_CACHE[B]


def _wdsac_block(pts_ref, xt_ref, tgt_ref, w1_ref, b1_ref, w2_ref, b2_ref,
                 sel1_ref, sel2_ref, sel3_ref, scal_ref, gpts_ref):
    f32 = jnp.float32
    hp = jax.lax.Precision.HIGHEST
    x3n = pts_ref[0]
    x2 = xt_ref[0]
    N = x2.shape[0]

    h = jnp.dot(x2, w1_ref[...], preferred_element_type=f32)
    h = jnp.maximum(h + b1_ref[...], 0.0)
    pw = jnp.dot(h, w2_ref[...], preferred_element_type=f32) + b2_ref[0, 0]
    pw_row = jnp.transpose(pw, (1, 0))

    jj = jax.lax.broadcasted_iota(jnp.int32, (N, N), 0)
    ii = jax.lax.broadcasted_iota(jnp.int32, (N, N), 1)
    cmp = (pw > pw_row) | ((pw == pw_row) & (jj < ii))
    rank = jnp.sum(cmp.astype(jnp.int32), axis=0, keepdims=True)

    r_iota = jax.lax.broadcasted_iota(jnp.int32, (_NUM_GPTS, N), 0)
    onehot = (rank == r_iota).astype(f32)
    gpts = jnp.dot(onehot, x2, precision=hp,
                   preferred_element_type=f32)
    gpts_ref[0] = gpts

    hr_iota = jax.lax.broadcasted_iota(jnp.int32, (_HYPS, _NUM_GPTS), 1)
    p = []
    for sel_ref in (sel1_ref, sel2_ref, sel3_ref):
        sel = sel_ref[0]
        oh = (sel == hr_iota).astype(f32)
        p.append(jnp.dot(oh, gpts, precision=hp,
                         preferred_element_type=f32))
    p1, p2, p3 = p

    u = p2 - p1
    v = p3 - p1
    ux, uy, uz = u[:, 0:1], u[:, 1:2], u[:, 2:3]
    vx, vy, vz = v[:, 0:1], v[:, 1:2], v[:, 2:3]
    nx = uy * vz - uz * vy
    ny = uz * vx - ux * vz
    nz = ux * vy - uy * vx
    dd = -(nx * p1[:, 0:1] + ny * p1[:, 1:2] + nz * p1[:, 2:3])
    dege = (nx == 0.0) & (ny == 0.0) & (nz == 0.0) & (dd == 0.0)
    one = jnp.ones_like(nx)
    nx = jnp.where(dege, one, nx)
    ny = jnp.where(dege, one, ny)
    nz = jnp.where(dege, one, nz)
    dd = jnp.where(dege, one, dd)
    norm = jnp.sqrt(nx ** 2 + ny ** 2 + nz ** 2)
    nx, ny, nz, dd = nx / norm, ny / norm, nz / norm, dd / norm

    n3 = jnp.concatenate([nx, ny, nz], axis=1)
    dist = jnp.dot(n3, x3n, preferred_element_type=f32) + dd
    score = jnp.sum(jnp.exp(-(dist * dist) / (2.0 * _INLIER_SIGMA2)),
                    axis=1, keepdims=True)

    tgt = tgt_ref[0]
    tx, ty, tz = tgt[:, 0:1], tgt[:, 1:2], tgt[:, 2:3]
    lm = (nx - tx) ** 2 + (ny - ty) ** 2 + (nz - tz) ** 2
    lp = (nx + tx) ** 2 + (ny + ty) ** 2 + (nz + tz) ** 2
    loss = jnp.minimum(lm, lp)

    ms = jnp.max(score, axis=0, keepdims=True)
    hi = jax.lax.broadcasted_iota(jnp.int32, (_HYPS, 1), 0).astype(f32)
    cand = jnp.where(score == ms, hi, f32(_HYPS))
    mi = jnp.min(cand, axis=0, keepdims=True)
    mh = (hi == mi).astype(f32)
    top_loss = jnp.sum(loss * mh, axis=0, keepdims=True)
    preds = [jnp.sum(n_ * mh, axis=0, keepdims=True)
             for n_ in (nx, ny, nz)]

    z = _INLIER_ALPHA * score
    zmax = jnp.max(z, axis=0, keepdims=True)
    e = jnp.exp(z - zmax)
    sm = e / jnp.sum(e, axis=0, keepdims=True)
    exp_loss = jnp.sum(loss * sm, axis=0, keepdims=True)

    zero = jnp.zeros_like(exp_loss)
    scal_ref[0] = jnp.concatenate(
        [exp_loss, top_loss, preds[0], preds[1], preds[2], zero, zero, zero],
        axis=1)


def kernel(pts, target, W1, b1, W2, b2):
    B, _, N = pts.shape
    sel1, sel2, sel3 = _get_sel(B)
    xt = jnp.transpose(pts, (0, 2, 1))
    grid = (B,)

    def bmap3(i):
        return (i, 0, 0)

    def cmap(i):
        return (0, 0)

    scal, gpts = pl.pallas_call(
        _wdsac_block,
        grid=grid,
        in_specs=[
            pl.BlockSpec((1, 3, N), bmap3),
            pl.BlockSpec((1, N, 3), bmap3),
            pl.BlockSpec((1, 1, 3), bmap3),
            pl.BlockSpec((3, 64), cmap),
            pl.BlockSpec((1, 64), cmap),
            pl.BlockSpec((64, 1), cmap),
            pl.BlockSpec((1, 1), cmap),
            pl.BlockSpec((1, _HYPS, 1), bmap3),
            pl.BlockSpec((1, _HYPS, 1), bmap3),
            pl.BlockSpec((1, _HYPS, 1), bmap3),
        ],
        out_specs=[
            pl.BlockSpec((1, 1, 8), bmap3),
            pl.BlockSpec((1, _NUM_GPTS, 3), bmap3),
        ],
        out_shape=[
            jax.ShapeDtypeStruct((B, 1, 8), jnp.float32),
            jax.ShapeDtypeStruct((B, _NUM_GPTS, 3), jnp.float32),
        ],
    )(pts, xt, target.reshape(B, 1, 3), W1, b1.reshape(1, 64), W2,
      b2.reshape(1, 1),
      jnp.asarray(sel1).reshape(B, _HYPS, 1),
      jnp.asarray(sel2).reshape(B, _HYPS, 1),
      jnp.asarray(sel3).reshape(B, _HYPS, 1))

    exp_loss = scal[:, 0, 0]
    top_loss = scal[:, 0, 1]
    pred = scal[:, 0, 2:5]
    return (exp_loss, top_loss, pred, gpts)


# speedup vs baseline: 1.1116x; 1.1116x over previous
import itertools

import jax
import jax.numpy as jnp
import numpy as np
from jax.experimental import pallas as pl

_NUM_GPTS = 128
_HYPS = 64
_INLIER_SIGMA2 = 0.01
_INLIER_ALPHA = 0.5

_SEL_CACHE = {}


def _threefry2x32(ks0, ks1, x0, x1):
    with np.errstate(over="ignore"):
        rot = ((13, 15, 26, 6), (17, 29, 16, 24))
        x0 = x0.astype(np.uint32).copy()
        x1 = x1.astype(np.uint32).copy()
        ks0 = np.uint32(ks0)
        ks1 = np.uint32(ks1)
        ks2 = np.uint32(ks0 ^ ks1 ^ np.uint32(0x1BD11BDA))
        x0 = x0 + ks0
        x1 = x1 + ks1
        inject = [(ks1, ks2, 1), (ks2, ks0, 2), (ks0, ks1, 3),
                  (ks1, ks2, 4), (ks2, ks0, 5)]
        for i in range(5):
            for d in rot[i % 2]:
                x0 = x0 + x1
                x1 = (x1 << np.uint32(d)) | (x1 >> np.uint32(32 - d))
                x1 = x1 ^ x0
            a, b, c = inject[i]
            x0 = x0 + a
            x1 = x1 + b + np.uint32(c)
        return x0, x1


def _np_randint_key42(size, maxval):
    with np.errstate(over="ignore"):
        b1, b2 = _threefry2x32(np.uint32(0), np.uint32(42),
                               np.zeros(2, np.uint32),
                               np.arange(2, dtype=np.uint32))
        k1 = (b1[0], b2[0])
        k2 = (b1[1], b2[1])
        lo = np.arange(size, dtype=np.uint32)
        hi = np.zeros(size, np.uint32)
        h1, h2 = _threefry2x32(k1[0], k1[1], hi, lo)
        higher = h1 ^ h2
        l1, l2 = _threefry2x32(k2[0], k2[1], hi, lo)
        lower = l1 ^ l2
        span = np.uint32(maxval)
        multiplier = np.uint32((2 ** 16) % maxval)
        multiplier = (multiplier * multiplier) % span
        off = (higher % span) * multiplier + (lower % span)
        off = off % span
        return off.astype(np.int32)


def _get_sel(B):
    if B not in _SEL_CACHE:
        idx_combi = np.array(
            list(itertools.combinations(range(_NUM_GPTS), 3)), dtype=np.int32)
        tmp = _np_randint_key42(_HYPS * B, idx_combi.shape[0])
        combos = idx_combi[tmp].reshape(B, _HYPS, 3)
        _SEL_CACHE[B] = (np.ascontiguousarray(combos[:, :, 0]),
                         np.ascontiguousarray(combos[:, :, 1]),
                         np.ascontiguousarray(combos[:, :, 2]))
    return _SEL_CACHE[B]


_UNROLL = 2


def _wdsac_block(pts_ref, xt_ref, tgt_ref, w1_ref, b1_ref, w2_ref, b2_ref,
                 sel1_ref, sel2_ref, sel3_ref, scal_ref, gpts_ref):
    for k in range(_UNROLL):
        _wdsac_one(k, pts_ref, xt_ref, tgt_ref, w1_ref, b1_ref, w2_ref,
                   b2_ref, sel1_ref, sel2_ref, sel3_ref, scal_ref, gpts_ref)


def _wdsac_one(k, pts_ref, xt_ref, tgt_ref, w1_ref, b1_ref, w2_ref, b2_ref,
               sel1_ref, sel2_ref, sel3_ref, scal_ref, gpts_ref):
    f32 = jnp.float32
    hp = jax.lax.Precision.HIGHEST
    x3n = pts_ref[k]
    x2 = xt_ref[k]
    N = x2.shape[0]

    h = jnp.dot(x2, w1_ref[...], preferred_element_type=f32)
    h = jnp.maximum(h + b1_ref[...], 0.0)
    pw = jnp.dot(h, w2_ref[...], preferred_element_type=f32) + b2_ref[0, 0]
    pw_row = jnp.transpose(pw, (1, 0))

    jj = jax.lax.broadcasted_iota(jnp.int32, (N, N), 0)
    ii = jax.lax.broadcasted_iota(jnp.int32, (N, N), 1)
    cmp = (pw > pw_row) | ((pw == pw_row) & (jj < ii))
    rank = jnp.sum(cmp.astype(jnp.int32), axis=0, keepdims=True)

    r_iota = jax.lax.broadcasted_iota(jnp.int32, (_NUM_GPTS, N), 0)
    onehot = (rank == r_iota).astype(f32)
    gpts = jnp.dot(onehot, x2, precision=hp,
                   preferred_element_type=f32)
    gpts_ref[k] = gpts

    hr_iota = jax.lax.broadcasted_iota(jnp.int32, (_HYPS, _NUM_GPTS), 1)
    p = []
    for sel_ref in (sel1_ref, sel2_ref, sel3_ref):
        sel = sel_ref[k]
        oh = (sel == hr_iota).astype(f32)
        p.append(jnp.dot(oh, gpts, precision=hp,
                         preferred_element_type=f32))
    p1, p2, p3 = p

    u = p2 - p1
    v = p3 - p1
    ux, uy, uz = u[:, 0:1], u[:, 1:2], u[:, 2:3]
    vx, vy, vz = v[:, 0:1], v[:, 1:2], v[:, 2:3]
    nx = uy * vz - uz * vy
    ny = uz * vx - ux * vz
    nz = ux * vy - uy * vx
    dd = -(nx * p1[:, 0:1] + ny * p1[:, 1:2] + nz * p1[:, 2:3])
    dege = (nx == 0.0) & (ny == 0.0) & (nz == 0.0) & (dd == 0.0)
    one = jnp.ones_like(nx)
    nx = jnp.where(dege, one, nx)
    ny = jnp.where(dege, one, ny)
    nz = jnp.where(dege, one, nz)
    dd = jnp.where(dege, one, dd)
    norm = jnp.sqrt(nx ** 2 + ny ** 2 + nz ** 2)
    nx, ny, nz, dd = nx / norm, ny / norm, nz / norm, dd / norm

    n3 = jnp.concatenate([nx, ny, nz], axis=1)
    dist = jnp.dot(n3, x3n, preferred_element_type=f32) + dd
    score = jnp.sum(jnp.exp(-(dist * dist) / (2.0 * _INLIER_SIGMA2)),
                    axis=1, keepdims=True)

    tgt = tgt_ref[k]
    tx, ty, tz = tgt[:, 0:1], tgt[:, 1:2], tgt[:, 2:3]
    lm = (nx - tx) ** 2 + (ny - ty) ** 2 + (nz - tz) ** 2
    lp = (nx + tx) ** 2 + (ny + ty) ** 2 + (nz + tz) ** 2
    loss = jnp.minimum(lm, lp)

    ms = jnp.max(score, axis=0, keepdims=True)
    hi = jax.lax.broadcasted_iota(jnp.int32, (_HYPS, 1), 0).astype(f32)
    cand = jnp.where(score == ms, hi, f32(_HYPS))
    mi = jnp.min(cand, axis=0, keepdims=True)
    mh = (hi == mi).astype(f32)
    top_loss = jnp.sum(loss * mh, axis=0, keepdims=True)
    preds = [jnp.sum(n_ * mh, axis=0, keepdims=True)
             for n_ in (nx, ny, nz)]

    z = _INLIER_ALPHA * score
    zmax = jnp.max(z, axis=0, keepdims=True)
    e = jnp.exp(z - zmax)
    sm = e / jnp.sum(e, axis=0, keepdims=True)
    exp_loss = jnp.sum(loss * sm, axis=0, keepdims=True)

    zero = jnp.zeros_like(exp_loss)
    scal_ref[k] = jnp.concatenate(
        [exp_loss, top_loss, preds[0], preds[1], preds[2], zero, zero, zero],
        axis=1)


def kernel(pts, target, W1, b1, W2, b2):
    B, _, N = pts.shape
    sel1, sel2, sel3 = _get_sel(B)
    xt = jnp.transpose(pts, (0, 2, 1))
    grid = (B // _UNROLL,)

    def bmap3(i):
        return (i, 0, 0)

    def cmap(i):
        return (0, 0)

    scal, gpts = pl.pallas_call(
        _wdsac_block,
        grid=grid,
        in_specs=[
            pl.BlockSpec((_UNROLL, 3, N), bmap3),
            pl.BlockSpec((_UNROLL, N, 3), bmap3),
            pl.BlockSpec((_UNROLL, 1, 3), bmap3),
            pl.BlockSpec((3, 64), cmap),
            pl.BlockSpec((1, 64), cmap),
            pl.BlockSpec((64, 1), cmap),
            pl.BlockSpec((1, 1), cmap),
            pl.BlockSpec((_UNROLL, _HYPS, 1), bmap3),
            pl.BlockSpec((_UNROLL, _HYPS, 1), bmap3),
            pl.BlockSpec((_UNROLL, _HYPS, 1), bmap3),
        ],
        out_specs=[
            pl.BlockSpec((_UNROLL, 1, 8), bmap3),
            pl.BlockSpec((_UNROLL, _NUM_GPTS, 3), bmap3),
        ],
        out_shape=[
            jax.ShapeDtypeStruct((B, 1, 8), jnp.float32),
            jax.ShapeDtypeStruct((B, _NUM_GPTS, 3), jnp.float32),
        ],
    )(pts, xt, target.reshape(B, 1, 3), W1, b1.reshape(1, 64), W2,
      b2.reshape(1, 1),
      jnp.asarray(sel1).reshape(B, _HYPS, 1),
      jnp.asarray(sel2).reshape(B, _HYPS, 1),
      jnp.asarray(sel3).reshape(B, _HYPS, 1))

    exp_loss = scal[:, 0, 0]
    top_loss = scal[:, 0, 1]
    pred = scal[:, 0, 2:5]
    return (exp_loss, top_loss, pred, gpts)


# speedup vs baseline: 1.3673x; 1.2301x over previous
import itertools

import jax
import jax.numpy as jnp
import numpy as np
from jax.experimental import pallas as pl

_NUM_GPTS = 128
_HYPS = 64
_INLIER_SIGMA2 = 0.01
_INLIER_ALPHA = 0.5

_SEL_CACHE = {}


def _threefry2x32(ks0, ks1, x0, x1):
    with np.errstate(over="ignore"):
        rot = ((13, 15, 26, 6), (17, 29, 16, 24))
        x0 = x0.astype(np.uint32).copy()
        x1 = x1.astype(np.uint32).copy()
        ks0 = np.uint32(ks0)
        ks1 = np.uint32(ks1)
        ks2 = np.uint32(ks0 ^ ks1 ^ np.uint32(0x1BD11BDA))
        x0 = x0 + ks0
        x1 = x1 + ks1
        inject = [(ks1, ks2, 1), (ks2, ks0, 2), (ks0, ks1, 3),
                  (ks1, ks2, 4), (ks2, ks0, 5)]
        for i in range(5):
            for d in rot[i % 2]:
                x0 = x0 + x1
                x1 = (x1 << np.uint32(d)) | (x1 >> np.uint32(32 - d))
                x1 = x1 ^ x0
            a, b, c = inject[i]
            x0 = x0 + a
            x1 = x1 + b + np.uint32(c)
        return x0, x1


def _np_randint_key42(size, maxval):
    with np.errstate(over="ignore"):
        b1, b2 = _threefry2x32(np.uint32(0), np.uint32(42),
                               np.zeros(2, np.uint32),
                               np.arange(2, dtype=np.uint32))
        k1 = (b1[0], b2[0])
        k2 = (b1[1], b2[1])
        lo = np.arange(size, dtype=np.uint32)
        hi = np.zeros(size, np.uint32)
        h1, h2 = _threefry2x32(k1[0], k1[1], hi, lo)
        higher = h1 ^ h2
        l1, l2 = _threefry2x32(k2[0], k2[1], hi, lo)
        lower = l1 ^ l2
        span = np.uint32(maxval)
        multiplier = np.uint32((2 ** 16) % maxval)
        multiplier = (multiplier * multiplier) % span
        off = (higher % span) * multiplier + (lower % span)
        off = off % span
        return off.astype(np.int32)


def _get_sel(B):
    if B not in _SEL_CACHE:
        idx_combi = np.array(
            list(itertools.combinations(range(_NUM_GPTS), 3)), dtype=np.int32)
        tmp = _np_randint_key42(_HYPS * B, idx_combi.shape[0])
        combos = idx_combi[tmp].reshape(B, _HYPS, 3)
        _SEL_CACHE[B] = (np.ascontiguousarray(combos[:, :, 0]),
                         np.ascontiguousarray(combos[:, :, 1]),
                         np.ascontiguousarray(combos[:, :, 2]))
    return _SEL_CACHE[B]


_UNROLL = 2


def _wdsac_block(pts_ref, xt_ref, tgt_ref, w1_ref, b1_ref, w2_ref, b2_ref,
                 sel1_ref, sel2_ref, sel3_ref, tri_ref, scal_ref, gpts_ref):
    for k in range(_UNROLL):
        _wdsac_one(k, pts_ref, xt_ref, tgt_ref, w1_ref, b1_ref, w2_ref,
                   b2_ref, sel1_ref, sel2_ref, sel3_ref, tri_ref, scal_ref,
                   gpts_ref)


def _wdsac_one(k, pts_ref, xt_ref, tgt_ref, w1_ref, b1_ref, w2_ref, b2_ref,
               sel1_ref, sel2_ref, sel3_ref, tri_ref, scal_ref, gpts_ref):
    f32 = jnp.float32
    hp = jax.lax.Precision.HIGHEST
    x3n = pts_ref[k]
    x2 = xt_ref[k]
    N = x2.shape[0]
    zero_f = jnp.zeros((), f32)
    one_f = jnp.ones((), f32)

    h = jnp.dot(x2, w1_ref[...], preferred_element_type=f32)
    h = jnp.maximum(h + b1_ref[...], 0.0)
    pw = jnp.dot(h, w2_ref[...], preferred_element_type=f32) + b2_ref[0, 0]
    pw_row = jnp.transpose(pw, (1, 0))

    gt = jnp.where(pw_row > pw, one_f, zero_f)
    eq = jnp.where(pw_row == pw, one_f, zero_f)
    cmp = gt + eq * tri_ref[...]
    rank = jnp.dot(cmp, jnp.ones((N, 1), f32),
                   preferred_element_type=f32)

    r_iota = jax.lax.broadcasted_iota(
        jnp.int32, (N, _NUM_GPTS), 1).astype(f32)
    oht = jnp.where(rank == r_iota, one_f, zero_f)
    gpts_t = jnp.dot(x3n, oht, precision=hp,
                     preferred_element_type=f32)
    gpts_ref[k] = gpts_t

    hr_iota = jax.lax.broadcasted_iota(jnp.int32, (_NUM_GPTS, _HYPS), 0)
    p = []
    for sel_ref in (sel1_ref, sel2_ref, sel3_ref):
        sel = sel_ref[k]
        oh = jnp.where(sel == hr_iota, one_f, zero_f)
        p.append(jnp.dot(gpts_t, oh, precision=hp,
                         preferred_element_type=f32))
    p1, p2, p3 = p

    u = p2 - p1
    v = p3 - p1
    ux, uy, uz = u[0:1, :], u[1:2, :], u[2:3, :]
    vx, vy, vz = v[0:1, :], v[1:2, :], v[2:3, :]
    nx = uy * vz - uz * vy
    ny = uz * vx - ux * vz
    nz = ux * vy - uy * vx
    dd = -(nx * p1[0:1, :] + ny * p1[1:2, :] + nz * p1[2:3, :])
    dege = (nx == 0.0) & (ny == 0.0) & (nz == 0.0) & (dd == 0.0)
    one = jnp.ones_like(nx)
    nx = jnp.where(dege, one, nx)
    ny = jnp.where(dege, one, ny)
    nz = jnp.where(dege, one, nz)
    dd = jnp.where(dege, one, dd)
    norm = jnp.sqrt(nx ** 2 + ny ** 2 + nz ** 2)
    nx, ny, nz, dd = nx / norm, ny / norm, nz / norm, dd / norm

    n3t = jnp.concatenate([nx, ny, nz], axis=0)
    dist = jnp.dot(x2, n3t, preferred_element_type=f32) + dd
    score = jnp.sum(jnp.exp(-(dist * dist) / (2.0 * _INLIER_SIGMA2)),
                    axis=0, keepdims=True)

    tgt = tgt_ref[k]
    tx, ty, tz = tgt[:, 0:1], tgt[:, 1:2], tgt[:, 2:3]
    lm = (nx - tx) ** 2 + (ny - ty) ** 2 + (nz - tz) ** 2
    lp = (nx + tx) ** 2 + (ny + ty) ** 2 + (nz + tz) ** 2
    loss = jnp.minimum(lm, lp)

    ms = jnp.max(score, axis=1, keepdims=True)
    hi = jax.lax.broadcasted_iota(jnp.int32, (1, _HYPS), 1).astype(f32)
    cand = jnp.where(score == ms, hi, f32(_HYPS))
    mi = jnp.min(cand, axis=1, keepdims=True)
    mh = jnp.where(hi == mi, one_f, zero_f)
    top_loss = jnp.sum(loss * mh, axis=1, keepdims=True)
    preds = [jnp.sum(n_ * mh, axis=1, keepdims=True)
             for n_ in (nx, ny, nz)]

    z = _INLIER_ALPHA * score
    zmax = jnp.max(z, axis=1, keepdims=True)
    e = jnp.exp(z - zmax)
    sm = e / jnp.sum(e, axis=1, keepdims=True)
    exp_loss = jnp.sum(loss * sm, axis=1, keepdims=True)

    zero = jnp.zeros_like(exp_loss)
    scal_ref[k] = jnp.concatenate(
        [exp_loss, top_loss, preds[0], preds[1], preds[2], zero, zero, zero],
        axis=1)


def kernel(pts, target, W1, b1, W2, b2):
    B, _, N = pts.shape
    sel1, sel2, sel3 = _get_sel(B)
    xt = jnp.transpose(pts, (0, 2, 1))
    tri = np.tril(np.ones((N, N), np.float32), -1)
    grid = (B // _UNROLL,)

    def bmap3(i):
        return (i, 0, 0)

    def cmap(i):
        return (0, 0)

    scal, gpts_t = pl.pallas_call(
        _wdsac_block,
        grid=grid,
        in_specs=[
            pl.BlockSpec((_UNROLL, 3, N), bmap3),
            pl.BlockSpec((_UNROLL, N, 3), bmap3),
            pl.BlockSpec((_UNROLL, 1, 3), bmap3),
            pl.BlockSpec((3, 64), cmap),
            pl.BlockSpec((1, 64), cmap),
            pl.BlockSpec((64, 1), cmap),
            pl.BlockSpec((1, 1), cmap),
            pl.BlockSpec((_UNROLL, 1, _HYPS), bmap3),
            pl.BlockSpec((_UNROLL, 1, _HYPS), bmap3),
            pl.BlockSpec((_UNROLL, 1, _HYPS), bmap3),
            pl.BlockSpec((N, N), cmap),
        ],
        out_specs=[
            pl.BlockSpec((_UNROLL, 1, 8), bmap3),
            pl.BlockSpec((_UNROLL, 3, _NUM_GPTS), bmap3),
        ],
        out_shape=[
            jax.ShapeDtypeStruct((B, 1, 8), jnp.float32),
            jax.ShapeDtypeStruct((B, 3, _NUM_GPTS), jnp.float32),
        ],
    )(pts, xt, target.reshape(B, 1, 3), W1, b1.reshape(1, 64), W2,
      b2.reshape(1, 1),
      jnp.asarray(sel1).reshape(B, 1, _HYPS),
      jnp.asarray(sel2).reshape(B, 1, _HYPS),
      jnp.asarray(sel3).reshape(B, 1, _HYPS),
      jnp.asarray(tri))

    exp_loss = scal[:, 0, 0]
    top_loss = scal[:, 0, 1]
    pred = scal[:, 0, 2:5]
    gpts = jnp.transpose(gpts_t, (0, 2, 1))
    return (exp_loss, top_loss, pred, gpts)


# speedup vs baseline: 1.4125x; 1.0330x over previous
import itertools

import jax
import jax.numpy as jnp
import numpy as np
from jax.experimental import pallas as pl

_NUM_GPTS = 128
_HYPS = 64
_INLIER_SIGMA2 = 0.01
_INLIER_ALPHA = 0.5

_SEL_CACHE = {}


def _threefry2x32(ks0, ks1, x0, x1):
    with np.errstate(over="ignore"):
        rot = ((13, 15, 26, 6), (17, 29, 16, 24))
        x0 = x0.astype(np.uint32).copy()
        x1 = x1.astype(np.uint32).copy()
        ks0 = np.uint32(ks0)
        ks1 = np.uint32(ks1)
        ks2 = np.uint32(ks0 ^ ks1 ^ np.uint32(0x1BD11BDA))
        x0 = x0 + ks0
        x1 = x1 + ks1
        inject = [(ks1, ks2, 1), (ks2, ks0, 2), (ks0, ks1, 3),
                  (ks1, ks2, 4), (ks2, ks0, 5)]
        for i in range(5):
            for d in rot[i % 2]:
                x0 = x0 + x1
                x1 = (x1 << np.uint32(d)) | (x1 >> np.uint32(32 - d))
                x1 = x1 ^ x0
            a, b, c = inject[i]
            x0 = x0 + a
            x1 = x1 + b + np.uint32(c)
        return x0, x1


def _np_randint_key42(size, maxval):
    with np.errstate(over="ignore"):
        b1, b2 = _threefry2x32(np.uint32(0), np.uint32(42),
                               np.zeros(2, np.uint32),
                               np.arange(2, dtype=np.uint32))
        k1 = (b1[0], b2[0])
        k2 = (b1[1], b2[1])
        lo = np.arange(size, dtype=np.uint32)
        hi = np.zeros(size, np.uint32)
        h1, h2 = _threefry2x32(k1[0], k1[1], hi, lo)
        higher = h1 ^ h2
        l1, l2 = _threefry2x32(k2[0], k2[1], hi, lo)
        lower = l1 ^ l2
        span = np.uint32(maxval)
        multiplier = np.uint32((2 ** 16) % maxval)
        multiplier = (multiplier * multiplier) % span
        off = (higher % span) * multiplier + (lower % span)
        off = off % span
        return off.astype(np.int32)


def _get_sel(B):
    if B not in _SEL_CACHE:
        idx_combi = np.array(
            list(itertools.combinations(range(_NUM_GPTS), 3)), dtype=np.int32)
        tmp = _np_randint_key42(_HYPS * B, idx_combi.shape[0])
        combos = idx_combi[tmp].reshape(B, _HYPS, 3)
        _SEL_CACHE[B] = (np.ascontiguousarray(combos[:, :, 0]),
                         np.ascontiguousarray(combos[:, :, 1]),
                         np.ascontiguousarray(combos[:, :, 2]))
    return _SEL_CACHE[B]


_UNROLL = 4


def _wdsac_block(pts_ref, xt_ref, tgt_ref, w1_ref, b1_ref, w2_ref, b2_ref,
                 sel1_ref, sel2_ref, sel3_ref, tri_ref, scal_ref, gpts_ref):
    for k in range(_UNROLL):
        _wdsac_one(k, pts_ref, xt_ref, tgt_ref, w1_ref, b1_ref, w2_ref,
                   b2_ref, sel1_ref, sel2_ref, sel3_ref, tri_ref, scal_ref,
                   gpts_ref)


def _wdsac_one(k, pts_ref, xt_ref, tgt_ref, w1_ref, b1_ref, w2_ref, b2_ref,
               sel1_ref, sel2_ref, sel3_ref, tri_ref, scal_ref, gpts_ref):
    f32 = jnp.float32
    hp = jax.lax.Precision.HIGHEST
    x3n = pts_ref[k]
    x2 = xt_ref[k]
    N = x2.shape[0]
    zero_f = jnp.zeros((), f32)
    one_f = jnp.ones((), f32)

    h = jnp.dot(x2, w1_ref[...], preferred_element_type=f32)
    h = jnp.maximum(h + b1_ref[...], 0.0)
    pw = jnp.dot(h, w2_ref[...], preferred_element_type=f32) + b2_ref[0, 0]
    pw_row = jnp.transpose(pw, (1, 0))

    cmp = jnp.where(pw_row > pw, one_f,
                    jnp.where(pw_row == pw, tri_ref[...], zero_f))
    rank = jnp.dot(cmp, jnp.ones((N, 1), f32),
                   preferred_element_type=f32)

    r_iota = jax.lax.broadcasted_iota(
        jnp.int32, (N, _NUM_GPTS), 1).astype(f32)
    oht = jnp.where(rank == r_iota, one_f, zero_f)
    gpts_t = jnp.dot(x3n, oht, precision=hp,
                     preferred_element_type=f32)
    gpts_ref[k] = gpts_t

    hr_iota = jax.lax.broadcasted_iota(jnp.int32, (_NUM_GPTS, _HYPS), 0)
    p = []
    for sel_ref in (sel1_ref, sel2_ref, sel3_ref):
        sel = sel_ref[k]
        oh = jnp.where(sel == hr_iota, one_f, zero_f)
        p.append(jnp.dot(gpts_t, oh, precision=hp,
                         preferred_element_type=f32))
    p1, p2, p3 = p

    u = p2 - p1
    v = p3 - p1
    ux, uy, uz = u[0:1, :], u[1:2, :], u[2:3, :]
    vx, vy, vz = v[0:1, :], v[1:2, :], v[2:3, :]
    nx = uy * vz - uz * vy
    ny = uz * vx - ux * vz
    nz = ux * vy - uy * vx
    dd = -(nx * p1[0:1, :] + ny * p1[1:2, :] + nz * p1[2:3, :])
    dege = (nx == 0.0) & (ny == 0.0) & (nz == 0.0) & (dd == 0.0)
    one = jnp.ones_like(nx)
    nx = jnp.where(dege, one, nx)
    ny = jnp.where(dege, one, ny)
    nz = jnp.where(dege, one, nz)
    dd = jnp.where(dege, one, dd)
    norm = jnp.sqrt(nx ** 2 + ny ** 2 + nz ** 2)
    nx, ny, nz, dd = nx / norm, ny / norm, nz / norm, dd / norm

    n3t = jnp.concatenate([nx, ny, nz], axis=0)
    dist = jnp.dot(x2, n3t, preferred_element_type=f32) + dd
    score = jnp.sum(jnp.exp(-(dist * dist) / (2.0 * _INLIER_SIGMA2)),
                    axis=0, keepdims=True)

    tgt = tgt_ref[k]
    tx, ty, tz = tgt[:, 0:1], tgt[:, 1:2], tgt[:, 2:3]
    lm = (nx - tx) ** 2 + (ny - ty) ** 2 + (nz - tz) ** 2
    lp = (nx + tx) ** 2 + (ny + ty) ** 2 + (nz + tz) ** 2
    loss = jnp.minimum(lm, lp)

    ms = jnp.max(score, axis=1, keepdims=True)
    hi = jax.lax.broadcasted_iota(jnp.int32, (1, _HYPS), 1).astype(f32)
    cand = jnp.where(score == ms, hi, f32(_HYPS))
    mi = jnp.min(cand, axis=1, keepdims=True)
    mh = jnp.where(hi == mi, one_f, zero_f)
    top_loss = jnp.sum(loss * mh, axis=1, keepdims=True)
    preds = [jnp.sum(n_ * mh, axis=1, keepdims=True)
             for n_ in (nx, ny, nz)]

    z = _INLIER_ALPHA * score
    zmax = jnp.max(z, axis=1, keepdims=True)
    e = jnp.exp(z - zmax)
    sm = e / jnp.sum(e, axis=1, keepdims=True)
    exp_loss = jnp.sum(loss * sm, axis=1, keepdims=True)

    zero = jnp.zeros_like(exp_loss)
    scal_ref[k] = jnp.concatenate(
        [exp_loss, top_loss, preds[0], preds[1], preds[2], zero, zero, zero],
        axis=1)


def kernel(pts, target, W1, b1, W2, b2):
    B, _, N = pts.shape
    sel1, sel2, sel3 = _get_sel(B)
    xt = jnp.transpose(pts, (0, 2, 1))
    tri = np.tril(np.ones((N, N), np.float32), -1)
    grid = (B // _UNROLL,)

    def bmap3(i):
        return (i, 0, 0)

    def cmap(i):
        return (0, 0)

    scal, gpts_t = pl.pallas_call(
        _wdsac_block,
        grid=grid,
        in_specs=[
            pl.BlockSpec((_UNROLL, 3, N), bmap3),
            pl.BlockSpec((_UNROLL, N, 3), bmap3),
            pl.BlockSpec((_UNROLL, 1, 3), bmap3),
            pl.BlockSpec((3, 64), cmap),
            pl.BlockSpec((1, 64), cmap),
            pl.BlockSpec((64, 1), cmap),
            pl.BlockSpec((1, 1), cmap),
            pl.BlockSpec((_UNROLL, 1, _HYPS), bmap3),
            pl.BlockSpec((_UNROLL, 1, _HYPS), bmap3),
            pl.BlockSpec((_UNROLL, 1, _HYPS), bmap3),
            pl.BlockSpec((N, N), cmap),
        ],
        out_specs=[
            pl.BlockSpec((_UNROLL, 1, 8), bmap3),
            pl.BlockSpec((_UNROLL, 3, _NUM_GPTS), bmap3),
        ],
        out_shape=[
            jax.ShapeDtypeStruct((B, 1, 8), jnp.float32),
            jax.ShapeDtypeStruct((B, 3, _NUM_GPTS), jnp.float32),
        ],
    )(pts, xt, target.reshape(B, 1, 3), W1, b1.reshape(1, 64), W2,
      b2.reshape(1, 1),
      jnp.asarray(sel1).reshape(B, 1, _HYPS),
      jnp.asarray(sel2).reshape(B, 1, _HYPS),
      jnp.asarray(sel3).reshape(B, 1, _HYPS),
      jnp.asarray(tri))

    exp_loss = scal[:, 0, 0]
    top_loss = scal[:, 0, 1]
    pred = scal[:, 0, 2:5]
    gpts = jnp.transpose(gpts_t, (0, 2, 1))
    return (exp_loss, top_loss, pred, gpts)


# speedup vs baseline: 2.7900x; 1.9752x over previous
import itertools

import jax
import jax.numpy as jnp
import numpy as np
from jax.experimental import pallas as pl

_NUM_GPTS = 128
_HYPS = 64
_INLIER_SIGMA2 = 0.01
_INLIER_ALPHA = 0.5

_SEL_CACHE = {}


def _threefry2x32(ks0, ks1, x0, x1):
    with np.errstate(over="ignore"):
        rot = ((13, 15, 26, 6), (17, 29, 16, 24))
        x0 = x0.astype(np.uint32).copy()
        x1 = x1.astype(np.uint32).copy()
        ks0 = np.uint32(ks0)
        ks1 = np.uint32(ks1)
        ks2 = np.uint32(ks0 ^ ks1 ^ np.uint32(0x1BD11BDA))
        x0 = x0 + ks0
        x1 = x1 + ks1
        inject = [(ks1, ks2, 1), (ks2, ks0, 2), (ks0, ks1, 3),
                  (ks1, ks2, 4), (ks2, ks0, 5)]
        for i in range(5):
            for d in rot[i % 2]:
                x0 = x0 + x1
                x1 = (x1 << np.uint32(d)) | (x1 >> np.uint32(32 - d))
                x1 = x1 ^ x0
            a, b, c = inject[i]
            x0 = x0 + a
            x1 = x1 + b + np.uint32(c)
        return x0, x1


def _np_randint_key42(size, maxval):
    with np.errstate(over="ignore"):
        b1, b2 = _threefry2x32(np.uint32(0), np.uint32(42),
                               np.zeros(2, np.uint32),
                               np.arange(2, dtype=np.uint32))
        k1 = (b1[0], b2[0])
        k2 = (b1[1], b2[1])
        lo = np.arange(size, dtype=np.uint32)
        hi = np.zeros(size, np.uint32)
        h1, h2 = _threefry2x32(k1[0], k1[1], hi, lo)
        higher = h1 ^ h2
        l1, l2 = _threefry2x32(k2[0], k2[1], hi, lo)
        lower = l1 ^ l2
        span = np.uint32(maxval)
        multiplier = np.uint32((2 ** 16) % maxval)
        multiplier = (multiplier * multiplier) % span
        off = (higher % span) * multiplier + (lower % span)
        off = off % span
        return off.astype(np.int32)


def _get_sel(B):
    if B not in _SEL_CACHE:
        idx_combi = np.array(
            list(itertools.combinations(range(_NUM_GPTS), 3)), dtype=np.int32)
        tmp = _np_randint_key42(_HYPS * B, idx_combi.shape[0])
        combos = idx_combi[tmp].reshape(B, _HYPS, 3)
        _SEL_CACHE[B] = (np.ascontiguousarray(combos[:, :, 0]),
                         np.ascontiguousarray(combos[:, :, 1]),
                         np.ascontiguousarray(combos[:, :, 2]))
    return _SEL_CACHE[B]


_UNROLL = 4


def _wdsac_block(pts_ref, xt_ref, tgt_ref, w1_ref, b1_ref, w2_ref, b2_ref,
                 sel1_ref, sel2_ref, sel3_ref, tri_ref, scal_ref, gpts_ref):
    U = _UNROLL
    f32 = jnp.float32
    hp = jax.lax.Precision.HIGHEST
    N = xt_ref.shape[1]
    one_f = jnp.ones((), f32)
    zero_f = jnp.zeros((), f32)

    x3 = [pts_ref[k] for k in range(U)]
    x2 = [xt_ref[k] for k in range(U)]
    w1 = w1_ref[...]
    b1 = b1_ref[...]
    w2 = w2_ref[...]
    b2 = b2_ref[0, 0]
    tri = tri_ref[...]

    hs = [jnp.maximum(jnp.dot(x2[k], w1, preferred_element_type=f32) + b1,
                      0.0) for k in range(U)]
    pw = [jnp.dot(hs[k], w2, preferred_element_type=f32) + b2
          for k in range(U)]
    pr = [jnp.transpose(pw[k], (1, 0)) for k in range(U)]

    cmps = [jnp.where(pr[k] > pw[k], one_f,
                      jnp.where(pr[k] == pw[k], tri, zero_f))
            for k in range(U)]
    ones_col = jnp.ones((N, 1), f32)
    ranks = [jnp.dot(cmps[k], ones_col, preferred_element_type=f32)
             for k in range(U)]

    r_iota = jax.lax.broadcasted_iota(
        jnp.int32, (N, _NUM_GPTS), 1).astype(f32)
    ohts = [jnp.where(ranks[k] == r_iota, one_f, zero_f) for k in range(U)]
    gpts_t = [jnp.dot(x3[k], ohts[k], precision=hp,
                      preferred_element_type=f32) for k in range(U)]
    for k in range(U):
        gpts_ref[k] = gpts_t[k]

    hr_iota = jax.lax.broadcasted_iota(jnp.int32, (_NUM_GPTS, _HYPS), 0)
    sels = (sel1_ref, sel2_ref, sel3_ref)
    ohs = [[jnp.where(sr[k] == hr_iota, one_f, zero_f) for sr in sels]
           for k in range(U)]
    ps = [[jnp.dot(gpts_t[k], ohs[k][j], precision=hp,
                   preferred_element_type=f32) for j in range(3)]
          for k in range(U)]

    nx_l, ny_l, nz_l, dd_l = [], [], [], []
    for k in range(U):
        p1, p2, p3 = ps[k]
        u = p2 - p1
        v = p3 - p1
        ux, uy, uz = u[0:1, :], u[1:2, :], u[2:3, :]
        vx, vy, vz = v[0:1, :], v[1:2, :], v[2:3, :]
        nx = uy * vz - uz * vy
        ny = uz * vx - ux * vz
        nz = ux * vy - uy * vx
        dd = -(nx * p1[0:1, :] + ny * p1[1:2, :] + nz * p1[2:3, :])
        nx_l.append(nx)
        ny_l.append(ny)
        nz_l.append(nz)
        dd_l.append(dd)
    NX = jnp.concatenate(nx_l, axis=0)
    NY = jnp.concatenate(ny_l, axis=0)
    NZ = jnp.concatenate(nz_l, axis=0)
    DD = jnp.concatenate(dd_l, axis=0)
    dege = (NX == 0.0) & (NY == 0.0) & (NZ == 0.0) & (DD == 0.0)
    one = jnp.ones_like(NX)
    NX = jnp.where(dege, one, NX)
    NY = jnp.where(dege, one, NY)
    NZ = jnp.where(dege, one, NZ)
    DD = jnp.where(dege, one, DD)
    norm = jnp.sqrt(NX ** 2 + NY ** 2 + NZ ** 2)
    NX, NY, NZ, DD = NX / norm, NY / norm, NZ / norm, DD / norm

    score_l = []
    for k in range(U):
        n3t = jnp.concatenate(
            [NX[k:k + 1, :], NY[k:k + 1, :], NZ[k:k + 1, :]], axis=0)
        dist = (jnp.dot(x2[k], n3t, preferred_element_type=f32)
                + DD[k:k + 1, :])
        score_l.append(jnp.sum(
            jnp.exp(-(dist * dist) / (2.0 * _INLIER_SIGMA2)),
            axis=0, keepdims=True))
    SCORE = jnp.concatenate(score_l, axis=0)

    tgt = tgt_ref[:, 0, :]
    tx, ty, tz = tgt[:, 0:1], tgt[:, 1:2], tgt[:, 2:3]
    lm = (NX - tx) ** 2 + (NY - ty) ** 2 + (NZ - tz) ** 2
    lp = (NX + tx) ** 2 + (NY + ty) ** 2 + (NZ + tz) ** 2
    LOSS = jnp.minimum(lm, lp)

    ms = jnp.max(SCORE, axis=1, keepdims=True)
    hi = jax.lax.broadcasted_iota(jnp.int32, (1, _HYPS), 1).astype(f32)
    cand = jnp.where(SCORE == ms, hi, f32(_HYPS))
    mi = jnp.min(cand, axis=1, keepdims=True)
    mh = jnp.where(hi == mi, one_f, zero_f)
    top_loss = jnp.sum(LOSS * mh, axis=1, keepdims=True)
    preds = [jnp.sum(n_ * mh, axis=1, keepdims=True)
             for n_ in (NX, NY, NZ)]

    z = _INLIER_ALPHA * SCORE
    zmax = jnp.max(z, axis=1, keepdims=True)
    e = jnp.exp(z - zmax)
    sm = e / jnp.sum(e, axis=1, keepdims=True)
    exp_loss = jnp.sum(LOSS * sm, axis=1, keepdims=True)

    zero = jnp.zeros((U, 3), f32)
    scal_ref[:, 0, :] = jnp.concatenate(
        [exp_loss, top_loss, preds[0], preds[1], preds[2], zero], axis=1)


def kernel(pts, target, W1, b1, W2, b2):
    B, _, N = pts.shape
    sel1, sel2, sel3 = _get_sel(B)
    xt = jnp.transpose(pts, (0, 2, 1))
    tri = np.tril(np.ones((N, N), np.float32), -1)
    grid = (B // _UNROLL,)

    def bmap3(i):
        return (i, 0, 0)

    def cmap(i):
        return (0, 0)

    scal, gpts_t = pl.pallas_call(
        _wdsac_block,
        grid=grid,
        in_specs=[
            pl.BlockSpec((_UNROLL, 3, N), bmap3),
            pl.BlockSpec((_UNROLL, N, 3), bmap3),
            pl.BlockSpec((_UNROLL, 1, 3), bmap3),
            pl.BlockSpec((3, 64), cmap),
            pl.BlockSpec((1, 64), cmap),
            pl.BlockSpec((64, 1), cmap),
            pl.BlockSpec((1, 1), cmap),
            pl.BlockSpec((_UNROLL, 1, _HYPS), bmap3),
            pl.BlockSpec((_UNROLL, 1, _HYPS), bmap3),
            pl.BlockSpec((_UNROLL, 1, _HYPS), bmap3),
            pl.BlockSpec((N, N), cmap),
        ],
        out_specs=[
            pl.BlockSpec((_UNROLL, 1, 8), bmap3),
            pl.BlockSpec((_UNROLL, 3, _NUM_GPTS), bmap3),
        ],
        out_shape=[
            jax.ShapeDtypeStruct((B, 1, 8), jnp.float32),
            jax.ShapeDtypeStruct((B, 3, _NUM_GPTS), jnp.float32),
        ],
    )(pts, xt, target.reshape(B, 1, 3), W1, b1.reshape(1, 64), W2,
      b2.reshape(1, 1),
      jnp.asarray(sel1).reshape(B, 1, _HYPS),
      jnp.asarray(sel2).reshape(B, 1, _HYPS),
      jnp.asarray(sel3).reshape(B, 1, _HYPS),
      jnp.asarray(tri))

    exp_loss = scal[:, 0, 0]
    top_loss = scal[:, 0, 1]
    pred = scal[:, 0, 2:5]
    gpts = jnp.transpose(gpts_t, (0, 2, 1))
    return (exp_loss, top_loss, pred, gpts)


# speedup vs baseline: 3.4403x; 1.2331x over previous
import itertools

import jax
import jax.numpy as jnp
import numpy as np
from jax.experimental import pallas as pl

_NUM_GPTS = 128
_HYPS = 64
_INLIER_SIGMA2 = 0.01
_INLIER_ALPHA = 0.5

_SEL_CACHE = {}


def _threefry2x32(ks0, ks1, x0, x1):
    with np.errstate(over="ignore"):
        rot = ((13, 15, 26, 6), (17, 29, 16, 24))
        x0 = x0.astype(np.uint32).copy()
        x1 = x1.astype(np.uint32).copy()
        ks0 = np.uint32(ks0)
        ks1 = np.uint32(ks1)
        ks2 = np.uint32(ks0 ^ ks1 ^ np.uint32(0x1BD11BDA))
        x0 = x0 + ks0
        x1 = x1 + ks1
        inject = [(ks1, ks2, 1), (ks2, ks0, 2), (ks0, ks1, 3),
                  (ks1, ks2, 4), (ks2, ks0, 5)]
        for i in range(5):
            for d in rot[i % 2]:
                x0 = x0 + x1
                x1 = (x1 << np.uint32(d)) | (x1 >> np.uint32(32 - d))
                x1 = x1 ^ x0
            a, b, c = inject[i]
            x0 = x0 + a
            x1 = x1 + b + np.uint32(c)
        return x0, x1


def _np_randint_key42(size, maxval):
    with np.errstate(over="ignore"):
        b1, b2 = _threefry2x32(np.uint32(0), np.uint32(42),
                               np.zeros(2, np.uint32),
                               np.arange(2, dtype=np.uint32))
        k1 = (b1[0], b2[0])
        k2 = (b1[1], b2[1])
        lo = np.arange(size, dtype=np.uint32)
        hi = np.zeros(size, np.uint32)
        h1, h2 = _threefry2x32(k1[0], k1[1], hi, lo)
        higher = h1 ^ h2
        l1, l2 = _threefry2x32(k2[0], k2[1], hi, lo)
        lower = l1 ^ l2
        span = np.uint32(maxval)
        multiplier = np.uint32((2 ** 16) % maxval)
        multiplier = (multiplier * multiplier) % span
        off = (higher % span) * multiplier + (lower % span)
        off = off % span
        return off.astype(np.int32)


def _get_sel(B):
    if B not in _SEL_CACHE:
        idx_combi = np.array(
            list(itertools.combinations(range(_NUM_GPTS), 3)), dtype=np.int32)
        tmp = _np_randint_key42(_HYPS * B, idx_combi.shape[0])
        combos = idx_combi[tmp].reshape(B, _HYPS, 3)
        _SEL_CACHE[B] = (np.ascontiguousarray(combos[:, :, 0]),
                         np.ascontiguousarray(combos[:, :, 1]),
                         np.ascontiguousarray(combos[:, :, 2]))
    return _SEL_CACHE[B]


_UNROLL = 8


def _wdsac_block(pts_ref, xt_ref, tgt_ref, w1_ref, b1_ref, w2_ref, b2_ref,
                 sel1_ref, sel2_ref, sel3_ref, tri_ref, scal_ref, gpts_ref):
    U = _UNROLL
    f32 = jnp.float32
    hp = jax.lax.Precision.HIGHEST
    N = xt_ref.shape[1]
    one_f = jnp.ones((), f32)
    zero_f = jnp.zeros((), f32)

    x3 = [pts_ref[k] for k in range(U)]
    x2 = [xt_ref[k] for k in range(U)]
    w1 = w1_ref[...]
    b1 = b1_ref[...]
    w2 = w2_ref[...]
    b2 = b2_ref[0, 0]
    tri = tri_ref[...]

    hs = [jnp.maximum(jnp.dot(x2[k], w1, preferred_element_type=f32) + b1,
                      0.0) for k in range(U)]
    pw = [jnp.dot(hs[k], w2, preferred_element_type=f32) + b2
          for k in range(U)]
    pr = [jnp.transpose(pw[k], (1, 0)) for k in range(U)]

    cmps = [jnp.where(pr[k] > pw[k], one_f,
                      jnp.where(pr[k] == pw[k], tri, zero_f))
            for k in range(U)]
    ones_col = jnp.ones((N, 1), f32)
    ranks = [jnp.dot(cmps[k], ones_col, preferred_element_type=f32)
             for k in range(U)]

    r_iota = jax.lax.broadcasted_iota(
        jnp.int32, (N, _NUM_GPTS), 1).astype(f32)
    ohts = [jnp.where(ranks[k] == r_iota, one_f, zero_f) for k in range(U)]
    gpts_t = [jnp.dot(x3[k], ohts[k], precision=hp,
                      preferred_element_type=f32) for k in range(U)]
    for k in range(U):
        gpts_ref[k] = gpts_t[k]

    hr_iota = jax.lax.broadcasted_iota(jnp.int32, (_NUM_GPTS, _HYPS), 0)
    sels = (sel1_ref, sel2_ref, sel3_ref)
    ohs = [[jnp.where(sr[k] == hr_iota, one_f, zero_f) for sr in sels]
           for k in range(U)]
    ps = [[jnp.dot(gpts_t[k], ohs[k][j], precision=hp,
                   preferred_element_type=f32) for j in range(3)]
          for k in range(U)]

    nx_l, ny_l, nz_l, dd_l = [], [], [], []
    for k in range(U):
        p1, p2, p3 = ps[k]
        u = p2 - p1
        v = p3 - p1
        ux, uy, uz = u[0:1, :], u[1:2, :], u[2:3, :]
        vx, vy, vz = v[0:1, :], v[1:2, :], v[2:3, :]
        nx = uy * vz - uz * vy
        ny = uz * vx - ux * vz
        nz = ux * vy - uy * vx
        dd = -(nx * p1[0:1, :] + ny * p1[1:2, :] + nz * p1[2:3, :])
        nx_l.append(nx)
        ny_l.append(ny)
        nz_l.append(nz)
        dd_l.append(dd)
    NX = jnp.concatenate(nx_l, axis=0)
    NY = jnp.concatenate(ny_l, axis=0)
    NZ = jnp.concatenate(nz_l, axis=0)
    DD = jnp.concatenate(dd_l, axis=0)
    dege = (NX == 0.0) & (NY == 0.0) & (NZ == 0.0) & (DD == 0.0)
    one = jnp.ones_like(NX)
    NX = jnp.where(dege, one, NX)
    NY = jnp.where(dege, one, NY)
    NZ = jnp.where(dege, one, NZ)
    DD = jnp.where(dege, one, DD)
    norm = jnp.sqrt(NX ** 2 + NY ** 2 + NZ ** 2)
    NX, NY, NZ, DD = NX / norm, NY / norm, NZ / norm, DD / norm

    score_l = []
    for k in range(U):
        n3t = jnp.concatenate(
            [NX[k:k + 1, :], NY[k:k + 1, :], NZ[k:k + 1, :]], axis=0)
        dist = (jnp.dot(x2[k], n3t, preferred_element_type=f32)
                + DD[k:k + 1, :])
        score_l.append(jnp.sum(
            jnp.exp(-(dist * dist) / (2.0 * _INLIER_SIGMA2)),
            axis=0, keepdims=True))
    SCORE = jnp.concatenate(score_l, axis=0)

    tgt = tgt_ref[:, 0, :]
    tx, ty, tz = tgt[:, 0:1], tgt[:, 1:2], tgt[:, 2:3]
    lm = (NX - tx) ** 2 + (NY - ty) ** 2 + (NZ - tz) ** 2
    lp = (NX + tx) ** 2 + (NY + ty) ** 2 + (NZ + tz) ** 2
    LOSS = jnp.minimum(lm, lp)

    ms = jnp.max(SCORE, axis=1, keepdims=True)
    hi = jax.lax.broadcasted_iota(jnp.int32, (1, _HYPS), 1).astype(f32)
    cand = jnp.where(SCORE == ms, hi, f32(_HYPS))
    mi = jnp.min(cand, axis=1, keepdims=True)
    mh = jnp.where(hi == mi, one_f, zero_f)
    top_loss = jnp.sum(LOSS * mh, axis=1, keepdims=True)
    preds = [jnp.sum(n_ * mh, axis=1, keepdims=True)
             for n_ in (NX, NY, NZ)]

    z = _INLIER_ALPHA * SCORE
    zmax = jnp.max(z, axis=1, keepdims=True)
    e = jnp.exp(z - zmax)
    sm = e / jnp.sum(e, axis=1, keepdims=True)
    exp_loss = jnp.sum(LOSS * sm, axis=1, keepdims=True)

    zero = jnp.zeros((U, 3), f32)
    scal_ref[:, 0, :] = jnp.concatenate(
        [exp_loss, top_loss, preds[0], preds[1], preds[2], zero], axis=1)


def kernel(pts, target, W1, b1, W2, b2):
    B, _, N = pts.shape
    sel1, sel2, sel3 = _get_sel(B)
    xt = jnp.transpose(pts, (0, 2, 1))
    tri = np.tril(np.ones((N, N), np.float32), -1)
    grid = (B // _UNROLL,)

    def bmap3(i):
        return (i, 0, 0)

    def cmap(i):
        return (0, 0)

    scal, gpts_t = pl.pallas_call(
        _wdsac_block,
        grid=grid,
        in_specs=[
            pl.BlockSpec((_UNROLL, 3, N), bmap3),
            pl.BlockSpec((_UNROLL, N, 3), bmap3),
            pl.BlockSpec((_UNROLL, 1, 3), bmap3),
            pl.BlockSpec((3, 64), cmap),
            pl.BlockSpec((1, 64), cmap),
            pl.BlockSpec((64, 1), cmap),
            pl.BlockSpec((1, 1), cmap),
            pl.BlockSpec((_UNROLL, 1, _HYPS), bmap3),
            pl.BlockSpec((_UNROLL, 1, _HYPS), bmap3),
            pl.BlockSpec((_UNROLL, 1, _HYPS), bmap3),
            pl.BlockSpec((N, N), cmap),
        ],
        out_specs=[
            pl.BlockSpec((_UNROLL, 1, 8), bmap3),
            pl.BlockSpec((_UNROLL, 3, _NUM_GPTS), bmap3),
        ],
        out_shape=[
            jax.ShapeDtypeStruct((B, 1, 8), jnp.float32),
            jax.ShapeDtypeStruct((B, 3, _NUM_GPTS), jnp.float32),
        ],
    )(pts, xt, target.reshape(B, 1, 3), W1, b1.reshape(1, 64), W2,
      b2.reshape(1, 1),
      jnp.asarray(sel1).reshape(B, 1, _HYPS),
      jnp.asarray(sel2).reshape(B, 1, _HYPS),
      jnp.asarray(sel3).reshape(B, 1, _HYPS),
      jnp.asarray(tri))

    exp_loss = scal[:, 0, 0]
    top_loss = scal[:, 0, 1]
    pred = scal[:, 0, 2:5]
    gpts = jnp.transpose(gpts_t, (0, 2, 1))
    return (exp_loss, top_loss, pred, gpts)


# speedup vs baseline: 3.9170x; 1.1386x over previous
import itertools

import jax
import jax.numpy as jnp
import numpy as np
from jax.experimental import pallas as pl

_NUM_GPTS = 128
_HYPS = 64
_INLIER_SIGMA2 = 0.01
_INLIER_ALPHA = 0.5

_SEL_CACHE = {}


def _threefry2x32(ks0, ks1, x0, x1):
    with np.errstate(over="ignore"):
        rot = ((13, 15, 26, 6), (17, 29, 16, 24))
        x0 = x0.astype(np.uint32).copy()
        x1 = x1.astype(np.uint32).copy()
        ks0 = np.uint32(ks0)
        ks1 = np.uint32(ks1)
        ks2 = np.uint32(ks0 ^ ks1 ^ np.uint32(0x1BD11BDA))
        x0 = x0 + ks0
        x1 = x1 + ks1
        inject = [(ks1, ks2, 1), (ks2, ks0, 2), (ks0, ks1, 3),
                  (ks1, ks2, 4), (ks2, ks0, 5)]
        for i in range(5):
            for d in rot[i % 2]:
                x0 = x0 + x1
                x1 = (x1 << np.uint32(d)) | (x1 >> np.uint32(32 - d))
                x1 = x1 ^ x0
            a, b, c = inject[i]
            x0 = x0 + a
            x1 = x1 + b + np.uint32(c)
        return x0, x1


def _np_randint_key42(size, maxval):
    with np.errstate(over="ignore"):
        b1, b2 = _threefry2x32(np.uint32(0), np.uint32(42),
                               np.zeros(2, np.uint32),
                               np.arange(2, dtype=np.uint32))
        k1 = (b1[0], b2[0])
        k2 = (b1[1], b2[1])
        lo = np.arange(size, dtype=np.uint32)
        hi = np.zeros(size, np.uint32)
        h1, h2 = _threefry2x32(k1[0], k1[1], hi, lo)
        higher = h1 ^ h2
        l1, l2 = _threefry2x32(k2[0], k2[1], hi, lo)
        lower = l1 ^ l2
        span = np.uint32(maxval)
        multiplier = np.uint32((2 ** 16) % maxval)
        multiplier = (multiplier * multiplier) % span
        off = (higher % span) * multiplier + (lower % span)
        off = off % span
        return off.astype(np.int32)


def _get_sel(B):
    if B not in _SEL_CACHE:
        idx_combi = np.array(
            list(itertools.combinations(range(_NUM_GPTS), 3)), dtype=np.int32)
        tmp = _np_randint_key42(_HYPS * B, idx_combi.shape[0])
        combos = idx_combi[tmp].reshape(B, _HYPS, 3)
        _SEL_CACHE[B] = (np.ascontiguousarray(combos[:, :, 0]),
                         np.ascontiguousarray(combos[:, :, 1]),
                         np.ascontiguousarray(combos[:, :, 2]))
    return _SEL_CACHE[B]


_UNROLL = 16


def _wdsac_block(pts_ref, xt_ref, tgt_ref, w1_ref, b1_ref, w2_ref, b2_ref,
                 sel1_ref, sel2_ref, sel3_ref, tri_ref, scal_ref, gpts_ref):
    U = _UNROLL
    f32 = jnp.float32
    hp = jax.lax.Precision.HIGHEST
    N = xt_ref.shape[1]
    one_f = jnp.ones((), f32)
    zero_f = jnp.zeros((), f32)

    x3 = [pts_ref[k] for k in range(U)]
    x2 = [xt_ref[k] for k in range(U)]
    w1 = w1_ref[...]
    b1 = b1_ref[...]
    w2 = w2_ref[...]
    b2 = b2_ref[0, 0]
    tri = tri_ref[...]

    hs = [jnp.maximum(jnp.dot(x2[k], w1, preferred_element_type=f32) + b1,
                      0.0) for k in range(U)]
    pw = [jnp.dot(hs[k], w2, preferred_element_type=f32) + b2
          for k in range(U)]
    pr = [jnp.transpose(pw[k], (1, 0)) for k in range(U)]

    cmps = [jnp.where(pr[k] > pw[k], one_f,
                      jnp.where(pr[k] == pw[k], tri, zero_f))
            for k in range(U)]
    ones_col = jnp.ones((N, 1), f32)
    ranks = [jnp.dot(cmps[k], ones_col, preferred_element_type=f32)
             for k in range(U)]

    r_iota = jax.lax.broadcasted_iota(
        jnp.int32, (N, _NUM_GPTS), 1).astype(f32)
    ohts = [jnp.where(ranks[k] == r_iota, one_f, zero_f) for k in range(U)]
    gpts_t = [jnp.dot(x3[k], ohts[k], precision=hp,
                      preferred_element_type=f32) for k in range(U)]
    for k in range(U):
        gpts_ref[k] = gpts_t[k]

    hr_iota = jax.lax.broadcasted_iota(jnp.int32, (_NUM_GPTS, _HYPS), 0)
    sels = (sel1_ref, sel2_ref, sel3_ref)
    ohs = [[jnp.where(sr[k] == hr_iota, one_f, zero_f) for sr in sels]
           for k in range(U)]
    ps = [[jnp.dot(gpts_t[k], ohs[k][j], precision=hp,
                   preferred_element_type=f32) for j in range(3)]
          for k in range(U)]

    nx_l, ny_l, nz_l, dd_l = [], [], [], []
    for k in range(U):
        p1, p2, p3 = ps[k]
        u = p2 - p1
        v = p3 - p1
        ux, uy, uz = u[0:1, :], u[1:2, :], u[2:3, :]
        vx, vy, vz = v[0:1, :], v[1:2, :], v[2:3, :]
        nx = uy * vz - uz * vy
        ny = uz * vx - ux * vz
        nz = ux * vy - uy * vx
        dd = -(nx * p1[0:1, :] + ny * p1[1:2, :] + nz * p1[2:3, :])
        nx_l.append(nx)
        ny_l.append(ny)
        nz_l.append(nz)
        dd_l.append(dd)
    NX = jnp.concatenate(nx_l, axis=0)
    NY = jnp.concatenate(ny_l, axis=0)
    NZ = jnp.concatenate(nz_l, axis=0)
    DD = jnp.concatenate(dd_l, axis=0)
    dege = (NX == 0.0) & (NY == 0.0) & (NZ == 0.0) & (DD == 0.0)
    one = jnp.ones_like(NX)
    NX = jnp.where(dege, one, NX)
    NY = jnp.where(dege, one, NY)
    NZ = jnp.where(dege, one, NZ)
    DD = jnp.where(dege, one, DD)
    norm = jnp.sqrt(NX ** 2 + NY ** 2 + NZ ** 2)
    NX, NY, NZ, DD = NX / norm, NY / norm, NZ / norm, DD / norm

    score_l = []
    for k in range(U):
        n3t = jnp.concatenate(
            [NX[k:k + 1, :], NY[k:k + 1, :], NZ[k:k + 1, :]], axis=0)
        dist = (jnp.dot(x2[k], n3t, preferred_element_type=f32)
                + DD[k:k + 1, :])
        score_l.append(jnp.sum(
            jnp.exp(-(dist * dist) / (2.0 * _INLIER_SIGMA2)),
            axis=0, keepdims=True))
    SCORE = jnp.concatenate(score_l, axis=0)

    tgt = tgt_ref[:, 0, :]
    tx, ty, tz = tgt[:, 0:1], tgt[:, 1:2], tgt[:, 2:3]
    lm = (NX - tx) ** 2 + (NY - ty) ** 2 + (NZ - tz) ** 2
    lp = (NX + tx) ** 2 + (NY + ty) ** 2 + (NZ + tz) ** 2
    LOSS = jnp.minimum(lm, lp)

    ms = jnp.max(SCORE, axis=1, keepdims=True)
    hi = jax.lax.broadcasted_iota(jnp.int32, (1, _HYPS), 1).astype(f32)
    cand = jnp.where(SCORE == ms, hi, f32(_HYPS))
    mi = jnp.min(cand, axis=1, keepdims=True)
    mh = jnp.where(hi == mi, one_f, zero_f)
    top_loss = jnp.sum(LOSS * mh, axis=1, keepdims=True)
    preds = [jnp.sum(n_ * mh, axis=1, keepdims=True)
             for n_ in (NX, NY, NZ)]

    z = _INLIER_ALPHA * SCORE
    zmax = jnp.max(z, axis=1, keepdims=True)
    e = jnp.exp(z - zmax)
    sm = e / jnp.sum(e, axis=1, keepdims=True)
    exp_loss = jnp.sum(LOSS * sm, axis=1, keepdims=True)

    zero = jnp.zeros((U, 3), f32)
    scal_ref[:, 0, :] = jnp.concatenate(
        [exp_loss, top_loss, preds[0], preds[1], preds[2], zero], axis=1)


def kernel(pts, target, W1, b1, W2, b2):
    B, _, N = pts.shape
    sel1, sel2, sel3 = _get_sel(B)
    xt = jnp.transpose(pts, (0, 2, 1))
    tri = np.tril(np.ones((N, N), np.float32), -1)
    grid = (B // _UNROLL,)

    def bmap3(i):
        return (i, 0, 0)

    def cmap(i):
        return (0, 0)

    scal, gpts_t = pl.pallas_call(
        _wdsac_block,
        grid=grid,
        in_specs=[
            pl.BlockSpec((_UNROLL, 3, N), bmap3),
            pl.BlockSpec((_UNROLL, N, 3), bmap3),
            pl.BlockSpec((_UNROLL, 1, 3), bmap3),
            pl.BlockSpec((3, 64), cmap),
            pl.BlockSpec((1, 64), cmap),
            pl.BlockSpec((64, 1), cmap),
            pl.BlockSpec((1, 1), cmap),
            pl.BlockSpec((_UNROLL, 1, _HYPS), bmap3),
            pl.BlockSpec((_UNROLL, 1, _HYPS), bmap3),
            pl.BlockSpec((_UNROLL, 1, _HYPS), bmap3),
            pl.BlockSpec((N, N), cmap),
        ],
        out_specs=[
            pl.BlockSpec((_UNROLL, 1, 8), bmap3),
            pl.BlockSpec((_UNROLL, 3, _NUM_GPTS), bmap3),
        ],
        out_shape=[
            jax.ShapeDtypeStruct((B, 1, 8), jnp.float32),
            jax.ShapeDtypeStruct((B, 3, _NUM_GPTS), jnp.float32),
        ],
    )(pts, xt, target.reshape(B, 1, 3), W1, b1.reshape(1, 64), W2,
      b2.reshape(1, 1),
      jnp.asarray(sel1).reshape(B, 1, _HYPS),
      jnp.asarray(sel2).reshape(B, 1, _HYPS),
      jnp.asarray(sel3).reshape(B, 1, _HYPS),
      jnp.asarray(tri))

    exp_loss = scal[:, 0, 0]
    top_loss = scal[:, 0, 1]
    pred = scal[:, 0, 2:5]
    gpts = jnp.transpose(gpts_t, (0, 2, 1))
    return (exp_loss, top_loss, pred, gpts)


# speedup vs baseline: 4.0518x; 1.0344x over previous
import itertools

import jax
import jax.numpy as jnp
import numpy as np
from jax.experimental import pallas as pl

_NUM_GPTS = 128
_HYPS = 64
_INLIER_SIGMA2 = 0.01
_INLIER_ALPHA = 0.5

_SEL_CACHE = {}


def _threefry2x32(ks0, ks1, x0, x1):
    with np.errstate(over="ignore"):
        rot = ((13, 15, 26, 6), (17, 29, 16, 24))
        x0 = x0.astype(np.uint32).copy()
        x1 = x1.astype(np.uint32).copy()
        ks0 = np.uint32(ks0)
        ks1 = np.uint32(ks1)
        ks2 = np.uint32(ks0 ^ ks1 ^ np.uint32(0x1BD11BDA))
        x0 = x0 + ks0
        x1 = x1 + ks1
        inject = [(ks1, ks2, 1), (ks2, ks0, 2), (ks0, ks1, 3),
                  (ks1, ks2, 4), (ks2, ks0, 5)]
        for i in range(5):
            for d in rot[i % 2]:
                x0 = x0 + x1
                x1 = (x1 << np.uint32(d)) | (x1 >> np.uint32(32 - d))
                x1 = x1 ^ x0
            a, b, c = inject[i]
            x0 = x0 + a
            x1 = x1 + b + np.uint32(c)
        return x0, x1


def _np_randint_key42(size, maxval):
    with np.errstate(over="ignore"):
        b1, b2 = _threefry2x32(np.uint32(0), np.uint32(42),
                               np.zeros(2, np.uint32),
                               np.arange(2, dtype=np.uint32))
        k1 = (b1[0], b2[0])
        k2 = (b1[1], b2[1])
        lo = np.arange(size, dtype=np.uint32)
        hi = np.zeros(size, np.uint32)
        h1, h2 = _threefry2x32(k1[0], k1[1], hi, lo)
        higher = h1 ^ h2
        l1, l2 = _threefry2x32(k2[0], k2[1], hi, lo)
        lower = l1 ^ l2
        span = np.uint32(maxval)
        multiplier = np.uint32((2 ** 16) % maxval)
        multiplier = (multiplier * multiplier) % span
        off = (higher % span) * multiplier + (lower % span)
        off = off % span
        return off.astype(np.int32)


def _get_sel(B):
    if B not in _SEL_CACHE:
        idx_combi = np.array(
            list(itertools.combinations(range(_NUM_GPTS), 3)), dtype=np.int32)
        tmp = _np_randint_key42(_HYPS * B, idx_combi.shape[0])
        combos = idx_combi[tmp].reshape(B, _HYPS, 3)
        _SEL_CACHE[B] = (np.ascontiguousarray(combos[:, :, 0]),
                         np.ascontiguousarray(combos[:, :, 1]),
                         np.ascontiguousarray(combos[:, :, 2]))
    return _SEL_CACHE[B]


_UNROLL = 32


def _wdsac_block(pts_ref, xt_ref, tgt_ref, w1_ref, b1_ref, w2_ref, b2_ref,
                 sel1_ref, sel2_ref, sel3_ref, tri_ref, scal_ref, gpts_ref):
    U = _UNROLL
    f32 = jnp.float32
    hp = jax.lax.Precision.HIGHEST
    N = xt_ref.shape[1]
    one_f = jnp.ones((), f32)
    zero_f = jnp.zeros((), f32)

    x3 = [pts_ref[k] for k in range(U)]
    x2 = [xt_ref[k] for k in range(U)]
    w1 = w1_ref[...]
    b1 = b1_ref[...]
    w2 = w2_ref[...]
    b2 = b2_ref[0, 0]
    tri = tri_ref[...]

    hs = [jnp.maximum(jnp.dot(x2[k], w1, preferred_element_type=f32) + b1,
                      0.0) for k in range(U)]
    pw = [jnp.dot(hs[k], w2, preferred_element_type=f32) + b2
          for k in range(U)]
    pr = [jnp.transpose(pw[k], (1, 0)) for k in range(U)]

    cmps = [jnp.where(pr[k] > pw[k], one_f,
                      jnp.where(pr[k] == pw[k], tri, zero_f))
            for k in range(U)]
    ones_col = jnp.ones((N, 1), f32)
    ranks = [jnp.dot(cmps[k], ones_col, preferred_element_type=f32)
             for k in range(U)]

    r_iota = jax.lax.broadcasted_iota(
        jnp.int32, (N, _NUM_GPTS), 1).astype(f32)
    ohts = [jnp.where(ranks[k] == r_iota, one_f, zero_f) for k in range(U)]
    gpts_t = [jnp.dot(x3[k], ohts[k], precision=hp,
                      preferred_element_type=f32) for k in range(U)]
    for k in range(U):
        gpts_ref[k] = gpts_t[k]

    hr_iota = jax.lax.broadcasted_iota(jnp.int32, (_NUM_GPTS, _HYPS), 0)
    sels = (sel1_ref, sel2_ref, sel3_ref)
    ohs = [[jnp.where(sr[k] == hr_iota, one_f, zero_f) for sr in sels]
           for k in range(U)]
    ps = [[jnp.dot(gpts_t[k], ohs[k][j], precision=hp,
                   preferred_element_type=f32) for j in range(3)]
          for k in range(U)]

    nx_l, ny_l, nz_l, dd_l = [], [], [], []
    for k in range(U):
        p1, p2, p3 = ps[k]
        u = p2 - p1
        v = p3 - p1
        ux, uy, uz = u[0:1, :], u[1:2, :], u[2:3, :]
        vx, vy, vz = v[0:1, :], v[1:2, :], v[2:3, :]
        nx = uy * vz - uz * vy
        ny = uz * vx - ux * vz
        nz = ux * vy - uy * vx
        dd = -(nx * p1[0:1, :] + ny * p1[1:2, :] + nz * p1[2:3, :])
        nx_l.append(nx)
        ny_l.append(ny)
        nz_l.append(nz)
        dd_l.append(dd)
    NX = jnp.concatenate(nx_l, axis=0)
    NY = jnp.concatenate(ny_l, axis=0)
    NZ = jnp.concatenate(nz_l, axis=0)
    DD = jnp.concatenate(dd_l, axis=0)
    dege = (NX == 0.0) & (NY == 0.0) & (NZ == 0.0) & (DD == 0.0)
    one = jnp.ones_like(NX)
    NX = jnp.where(dege, one, NX)
    NY = jnp.where(dege, one, NY)
    NZ = jnp.where(dege, one, NZ)
    DD = jnp.where(dege, one, DD)
    norm = jnp.sqrt(NX ** 2 + NY ** 2 + NZ ** 2)
    NX, NY, NZ, DD = NX / norm, NY / norm, NZ / norm, DD / norm

    score_l = []
    for k in range(U):
        n3t = jnp.concatenate(
            [NX[k:k + 1, :], NY[k:k + 1, :], NZ[k:k + 1, :]], axis=0)
        dist = (jnp.dot(x2[k], n3t, preferred_element_type=f32)
                + DD[k:k + 1, :])
        score_l.append(jnp.sum(
            jnp.exp(-(dist * dist) / (2.0 * _INLIER_SIGMA2)),
            axis=0, keepdims=True))
    SCORE = jnp.concatenate(score_l, axis=0)

    tgt = tgt_ref[:, 0, :]
    tx, ty, tz = tgt[:, 0:1], tgt[:, 1:2], tgt[:, 2:3]
    lm = (NX - tx) ** 2 + (NY - ty) ** 2 + (NZ - tz) ** 2
    lp = (NX + tx) ** 2 + (NY + ty) ** 2 + (NZ + tz) ** 2
    LOSS = jnp.minimum(lm, lp)

    ms = jnp.max(SCORE, axis=1, keepdims=True)
    hi = jax.lax.broadcasted_iota(jnp.int32, (1, _HYPS), 1).astype(f32)
    cand = jnp.where(SCORE == ms, hi, f32(_HYPS))
    mi = jnp.min(cand, axis=1, keepdims=True)
    mh = jnp.where(hi == mi, one_f, zero_f)
    top_loss = jnp.sum(LOSS * mh, axis=1, keepdims=True)
    preds = [jnp.sum(n_ * mh, axis=1, keepdims=True)
             for n_ in (NX, NY, NZ)]

    z = _INLIER_ALPHA * SCORE
    zmax = jnp.max(z, axis=1, keepdims=True)
    e = jnp.exp(z - zmax)
    sm = e / jnp.sum(e, axis=1, keepdims=True)
    exp_loss = jnp.sum(LOSS * sm, axis=1, keepdims=True)

    zero = jnp.zeros((U, 3), f32)
    scal_ref[:, 0, :] = jnp.concatenate(
        [exp_loss, top_loss, preds[0], preds[1], preds[2], zero], axis=1)


def kernel(pts, target, W1, b1, W2, b2):
    B, _, N = pts.shape
    sel1, sel2, sel3 = _get_sel(B)
    xt = jnp.transpose(pts, (0, 2, 1))
    tri = np.tril(np.ones((N, N), np.float32), -1)
    grid = (B // _UNROLL,)

    def bmap3(i):
        return (i, 0, 0)

    def cmap(i):
        return (0, 0)

    scal, gpts_t = pl.pallas_call(
        _wdsac_block,
        grid=grid,
        in_specs=[
            pl.BlockSpec((_UNROLL, 3, N), bmap3),
            pl.BlockSpec((_UNROLL, N, 3), bmap3),
            pl.BlockSpec((_UNROLL, 1, 3), bmap3),
            pl.BlockSpec((3, 64), cmap),
            pl.BlockSpec((1, 64), cmap),
            pl.BlockSpec((64, 1), cmap),
            pl.BlockSpec((1, 1), cmap),
            pl.BlockSpec((_UNROLL, 1, _HYPS), bmap3),
            pl.BlockSpec((_UNROLL, 1, _HYPS), bmap3),
            pl.BlockSpec((_UNROLL, 1, _HYPS), bmap3),
            pl.BlockSpec((N, N), cmap),
        ],
        out_specs=[
            pl.BlockSpec((_UNROLL, 1, 8), bmap3),
            pl.BlockSpec((_UNROLL, 3, _NUM_GPTS), bmap3),
        ],
        out_shape=[
            jax.ShapeDtypeStruct((B, 1, 8), jnp.float32),
            jax.ShapeDtypeStruct((B, 3, _NUM_GPTS), jnp.float32),
        ],
    )(pts, xt, target.reshape(B, 1, 3), W1, b1.reshape(1, 64), W2,
      b2.reshape(1, 1),
      jnp.asarray(sel1).reshape(B, 1, _HYPS),
      jnp.asarray(sel2).reshape(B, 1, _HYPS),
      jnp.asarray(sel3).reshape(B, 1, _HYPS),
      jnp.asarray(tri))

    exp_loss = scal[:, 0, 0]
    top_loss = scal[:, 0, 1]
    pred = scal[:, 0, 2:5]
    gpts = jnp.transpose(gpts_t, (0, 2, 1))
    return (exp_loss, top_loss, pred, gpts)


# speedup vs baseline: 5.0517x; 1.2468x over previous
import itertools

import jax
import jax.numpy as jnp
import numpy as np
from jax.experimental import pallas as pl

_NUM_GPTS = 128
_HYPS = 64
_INLIER_SIGMA2 = 0.01
_INLIER_ALPHA = 0.5

_SEL_CACHE = {}


def _threefry2x32(ks0, ks1, x0, x1):
    with np.errstate(over="ignore"):
        rot = ((13, 15, 26, 6), (17, 29, 16, 24))
        x0 = x0.astype(np.uint32).copy()
        x1 = x1.astype(np.uint32).copy()
        ks0 = np.uint32(ks0)
        ks1 = np.uint32(ks1)
        ks2 = np.uint32(ks0 ^ ks1 ^ np.uint32(0x1BD11BDA))
        x0 = x0 + ks0
        x1 = x1 + ks1
        inject = [(ks1, ks2, 1), (ks2, ks0, 2), (ks0, ks1, 3),
                  (ks1, ks2, 4), (ks2, ks0, 5)]
        for i in range(5):
            for d in rot[i % 2]:
                x0 = x0 + x1
                x1 = (x1 << np.uint32(d)) | (x1 >> np.uint32(32 - d))
                x1 = x1 ^ x0
            a, b, c = inject[i]
            x0 = x0 + a
            x1 = x1 + b + np.uint32(c)
        return x0, x1


def _np_randint_key42(size, maxval):
    with np.errstate(over="ignore"):
        b1, b2 = _threefry2x32(np.uint32(0), np.uint32(42),
                               np.zeros(2, np.uint32),
                               np.arange(2, dtype=np.uint32))
        k1 = (b1[0], b2[0])
        k2 = (b1[1], b2[1])
        lo = np.arange(size, dtype=np.uint32)
        hi = np.zeros(size, np.uint32)
        h1, h2 = _threefry2x32(k1[0], k1[1], hi, lo)
        higher = h1 ^ h2
        l1, l2 = _threefry2x32(k2[0], k2[1], hi, lo)
        lower = l1 ^ l2
        span = np.uint32(maxval)
        multiplier = np.uint32((2 ** 16) % maxval)
        multiplier = (multiplier * multiplier) % span
        off = (higher % span) * multiplier + (lower % span)
        off = off % span
        return off.astype(np.int32)


def _get_sel(B):
    if B not in _SEL_CACHE:
        idx_combi = np.array(
            list(itertools.combinations(range(_NUM_GPTS), 3)), dtype=np.int32)
        tmp = _np_randint_key42(_HYPS * B, idx_combi.shape[0])
        combos = idx_combi[tmp].reshape(B, _HYPS, 3)
        _SEL_CACHE[B] = np.ascontiguousarray(
            np.concatenate([combos[:, :, 0], combos[:, :, 1],
                            combos[:, :, 2]], axis=1))
    return _SEL_CACHE[B]


_UNROLL = 32


def _wdsac_block(pts_ref, xt_ref, tgt_ref, w1_ref, b1_ref, w2_ref, b2_ref,
                 sel_ref, tri_ref, scal_ref, gpts_ref):
    U = _UNROLL
    f32 = jnp.float32
    hp = jax.lax.Precision.HIGHEST
    N = xt_ref.shape[1]
    one_f = jnp.ones((), f32)
    zero_f = jnp.zeros((), f32)

    x3 = [pts_ref[k] for k in range(U)]
    x2 = [xt_ref[k] for k in range(U)]
    w1 = w1_ref[...]
    b1 = b1_ref[...]
    w2 = w2_ref[...]
    b2 = b2_ref[0, 0]
    tri = tri_ref[...]

    hs = [jnp.maximum(jnp.dot(x2[k], w1, preferred_element_type=f32) + b1,
                      0.0) for k in range(U)]
    pw = [jnp.dot(hs[k], w2, preferred_element_type=f32) + b2
          for k in range(U)]
    pr = [jnp.transpose(pw[k], (1, 0)) for k in range(U)]

    cmps = [jnp.where(pr[k] > pw[k], one_f,
                      jnp.where(pr[k] == pw[k], tri, zero_f))
            for k in range(U)]
    ones_col = jnp.ones((N, 1), f32)
    ranks = [jnp.dot(cmps[k], ones_col, preferred_element_type=f32)
             for k in range(U)]

    r_iota = jax.lax.broadcasted_iota(
        jnp.int32, (N, _NUM_GPTS), 1).astype(f32)
    ohts = [jnp.where(ranks[k] == r_iota, one_f, zero_f) for k in range(U)]
    gpts_t = [jnp.dot(x3[k], ohts[k], precision=hp,
                      preferred_element_type=f32) for k in range(U)]
    for k in range(U):
        gpts_ref[k] = gpts_t[k]

    hr_iota = jax.lax.broadcasted_iota(jnp.int32, (_NUM_GPTS, 3 * _HYPS), 0)
    ohs = [jnp.where(sel_ref[k] == hr_iota, one_f, zero_f)
           for k in range(U)]
    psc = [jnp.dot(gpts_t[k], ohs[k], precision=hp,
                   preferred_element_type=f32) for k in range(U)]
    ps = [[psc[k][:, 0:_HYPS], psc[k][:, _HYPS:2 * _HYPS],
           psc[k][:, 2 * _HYPS:3 * _HYPS]] for k in range(U)]

    nx_l, ny_l, nz_l, dd_l = [], [], [], []
    for k in range(U):
        p1, p2, p3 = ps[k]
        u = p2 - p1
        v = p3 - p1
        ux, uy, uz = u[0:1, :], u[1:2, :], u[2:3, :]
        vx, vy, vz = v[0:1, :], v[1:2, :], v[2:3, :]
        nx = uy * vz - uz * vy
        ny = uz * vx - ux * vz
        nz = ux * vy - uy * vx
        dd = -(nx * p1[0:1, :] + ny * p1[1:2, :] + nz * p1[2:3, :])
        nx_l.append(nx)
        ny_l.append(ny)
        nz_l.append(nz)
        dd_l.append(dd)
    NX = jnp.concatenate(nx_l, axis=0)
    NY = jnp.concatenate(ny_l, axis=0)
    NZ = jnp.concatenate(nz_l, axis=0)
    DD = jnp.concatenate(dd_l, axis=0)
    dege = (NX == 0.0) & (NY == 0.0) & (NZ == 0.0) & (DD == 0.0)
    one = jnp.ones_like(NX)
    NX = jnp.where(dege, one, NX)
    NY = jnp.where(dege, one, NY)
    NZ = jnp.where(dege, one, NZ)
    DD = jnp.where(dege, one, DD)
    norm = jnp.sqrt(NX ** 2 + NY ** 2 + NZ ** 2)
    NX, NY, NZ, DD = NX / norm, NY / norm, NZ / norm, DD / norm

    score_l = []
    for k in range(U):
        n3t = jnp.concatenate(
            [NX[k:k + 1, :], NY[k:k + 1, :], NZ[k:k + 1, :]], axis=0)
        dist = (jnp.dot(x2[k], n3t, preferred_element_type=f32)
                + DD[k:k + 1, :])
        score_l.append(jnp.sum(
            jnp.exp(-(dist * dist) / (2.0 * _INLIER_SIGMA2)),
            axis=0, keepdims=True))
    SCORE = jnp.concatenate(score_l, axis=0)

    tgt = tgt_ref[:, 0, :]
    tx, ty, tz = tgt[:, 0:1], tgt[:, 1:2], tgt[:, 2:3]
    lm = (NX - tx) ** 2 + (NY - ty) ** 2 + (NZ - tz) ** 2
    lp = (NX + tx) ** 2 + (NY + ty) ** 2 + (NZ + tz) ** 2
    LOSS = jnp.minimum(lm, lp)

    ms = jnp.max(SCORE, axis=1, keepdims=True)
    hi = jax.lax.broadcasted_iota(jnp.int32, (1, _HYPS), 1).astype(f32)
    cand = jnp.where(SCORE == ms, hi, f32(_HYPS))
    mi = jnp.min(cand, axis=1, keepdims=True)
    mh = jnp.where(hi == mi, one_f, zero_f)
    top_loss = jnp.sum(LOSS * mh, axis=1, keepdims=True)
    preds = [jnp.sum(n_ * mh, axis=1, keepdims=True)
             for n_ in (NX, NY, NZ)]

    z = _INLIER_ALPHA * SCORE
    zmax = jnp.max(z, axis=1, keepdims=True)
    e = jnp.exp(z - zmax)
    sm = e / jnp.sum(e, axis=1, keepdims=True)
    exp_loss = jnp.sum(LOSS * sm, axis=1, keepdims=True)

    zero = jnp.zeros((U, 3), f32)
    scal_ref[:, 0, :] = jnp.concatenate(
        [exp_loss, top_loss, preds[0], preds[1], preds[2], zero], axis=1)


def kernel(pts, target, W1, b1, W2, b2):
    B, _, N = pts.shape
    selc = _get_sel(B)
    xt = jnp.transpose(pts, (0, 2, 1))
    tri = np.tril(np.ones((N, N), np.float32), -1)
    grid = (B // _UNROLL,)

    def bmap3(i):
        return (i, 0, 0)

    def cmap(i):
        return (0, 0)

    scal, gpts_t = pl.pallas_call(
        _wdsac_block,
        grid=grid,
        in_specs=[
            pl.BlockSpec((_UNROLL, 3, N), bmap3),
            pl.BlockSpec((_UNROLL, N, 3), bmap3),
            pl.BlockSpec((_UNROLL, 1, 3), bmap3),
            pl.BlockSpec((3, 64), cmap),
            pl.BlockSpec((1, 64), cmap),
            pl.BlockSpec((64, 1), cmap),
            pl.BlockSpec((1, 1), cmap),
            pl.BlockSpec((_UNROLL, 1, 3 * _HYPS), bmap3),
            pl.BlockSpec((N, N), cmap),
        ],
        out_specs=[
            pl.BlockSpec((_UNROLL, 1, 8), bmap3),
            pl.BlockSpec((_UNROLL, 3, _NUM_GPTS), bmap3),
        ],
        out_shape=[
            jax.ShapeDtypeStruct((B, 1, 8), jnp.float32),
            jax.ShapeDtypeStruct((B, 3, _NUM_GPTS), jnp.float32),
        ],
    )(pts, xt, target.reshape(B, 1, 3), W1, b1.reshape(1, 64), W2,
      b2.reshape(1, 1),
      jnp.asarray(selc).reshape(B, 1, 3 * _HYPS),
      jnp.asarray(tri))

    exp_loss = scal[:, 0, 0]
    top_loss = scal[:, 0, 1]
    pred = scal[:, 0, 2:5]
    gpts = jnp.transpose(gpts_t, (0, 2, 1))
    return (exp_loss, top_loss, pred, gpts)


# speedup vs baseline: 5.2742x; 1.0440x over previous
import itertools

import jax
import jax.numpy as jnp
import numpy as np
from jax.experimental import pallas as pl

_NUM_GPTS = 128
_HYPS = 64
_INLIER_SIGMA2 = 0.01
_INLIER_ALPHA = 0.5

_SEL_CACHE = {}


def _threefry2x32(ks0, ks1, x0, x1):
    with np.errstate(over="ignore"):
        rot = ((13, 15, 26, 6), (17, 29, 16, 24))
        x0 = x0.astype(np.uint32).copy()
        x1 = x1.astype(np.uint32).copy()
        ks0 = np.uint32(ks0)
        ks1 = np.uint32(ks1)
        ks2 = np.uint32(ks0 ^ ks1 ^ np.uint32(0x1BD11BDA))
        x0 = x0 + ks0
        x1 = x1 + ks1
        inject = [(ks1, ks2, 1), (ks2, ks0, 2), (ks0, ks1, 3),
                  (ks1, ks2, 4), (ks2, ks0, 5)]
        for i in range(5):
            for d in rot[i % 2]:
                x0 = x0 + x1
                x1 = (x1 << np.uint32(d)) | (x1 >> np.uint32(32 - d))
                x1 = x1 ^ x0
            a, b, c = inject[i]
            x0 = x0 + a
            x1 = x1 + b + np.uint32(c)
        return x0, x1


def _np_randint_key42(size, maxval):
    with np.errstate(over="ignore"):
        b1, b2 = _threefry2x32(np.uint32(0), np.uint32(42),
                               np.zeros(2, np.uint32),
                               np.arange(2, dtype=np.uint32))
        k1 = (b1[0], b2[0])
        k2 = (b1[1], b2[1])
        lo = np.arange(size, dtype=np.uint32)
        hi = np.zeros(size, np.uint32)
        h1, h2 = _threefry2x32(k1[0], k1[1], hi, lo)
        higher = h1 ^ h2
        l1, l2 = _threefry2x32(k2[0], k2[1], hi, lo)
        lower = l1 ^ l2
        span = np.uint32(maxval)
        multiplier = np.uint32((2 ** 16) % maxval)
        multiplier = (multiplier * multiplier) % span
        off = (higher % span) * multiplier + (lower % span)
        off = off % span
        return off.astype(np.int32)


def _get_sel(B):
    if B not in _SEL_CACHE:
        idx_combi = np.array(
            list(itertools.combinations(range(_NUM_GPTS), 3)), dtype=np.int32)
        tmp = _np_randint_key42(_HYPS * B, idx_combi.shape[0])
        combos = idx_combi[tmp].reshape(B, _HYPS, 3)
        _SEL_CACHE[B] = np.ascontiguousarray(
            np.concatenate([combos[:, :, 0], combos[:, :, 1],
                            combos[:, :, 2]], axis=1))
    return _SEL_CACHE[B]


_UNROLL = 64


def _wdsac_block(pts_ref, xt_ref, tgt_ref, w1_ref, b1_ref, w2_ref, b2_ref,
                 sel_ref, tri_ref, scal_ref, gpts_ref):
    U = _UNROLL
    f32 = jnp.float32
    hp = jax.lax.Precision.HIGHEST
    N = xt_ref.shape[1]
    one_f = jnp.ones((), f32)
    zero_f = jnp.zeros((), f32)

    x3 = [pts_ref[k] for k in range(U)]
    x2 = [xt_ref[k] for k in range(U)]
    w1 = w1_ref[...]
    b1 = b1_ref[...]
    w2 = w2_ref[...]
    b2 = b2_ref[0, 0]
    tri = tri_ref[...]

    hs = [jnp.maximum(jnp.dot(x2[k], w1, preferred_element_type=f32) + b1,
                      0.0) for k in range(U)]
    pw = [jnp.dot(hs[k], w2, preferred_element_type=f32) + b2
          for k in range(U)]
    pr = [jnp.transpose(pw[k], (1, 0)) for k in range(U)]

    cmps = [jnp.where(pr[k] > pw[k], one_f,
                      jnp.where(pr[k] == pw[k], tri, zero_f))
            for k in range(U)]
    ones_col = jnp.ones((N, 1), f32)
    ranks = [jnp.dot(cmps[k], ones_col, preferred_element_type=f32)
             for k in range(U)]

    r_iota = jax.lax.broadcasted_iota(
        jnp.int32, (N, _NUM_GPTS), 1).astype(f32)
    ohts = [jnp.where(ranks[k] == r_iota, one_f, zero_f) for k in range(U)]
    gpts_t = [jnp.dot(x3[k], ohts[k], precision=hp,
                      preferred_element_type=f32) for k in range(U)]
    for k in range(U):
        gpts_ref[k] = gpts_t[k]

    hr_iota = jax.lax.broadcasted_iota(jnp.int32, (_NUM_GPTS, 3 * _HYPS), 0)
    ohs = [jnp.where(sel_ref[k] == hr_iota, one_f, zero_f)
           for k in range(U)]
    psc = [jnp.dot(gpts_t[k], ohs[k], precision=hp,
                   preferred_element_type=f32) for k in range(U)]
    ps = [[psc[k][:, 0:_HYPS], psc[k][:, _HYPS:2 * _HYPS],
           psc[k][:, 2 * _HYPS:3 * _HYPS]] for k in range(U)]

    nx_l, ny_l, nz_l, dd_l = [], [], [], []
    for k in range(U):
        p1, p2, p3 = ps[k]
        u = p2 - p1
        v = p3 - p1
        ux, uy, uz = u[0:1, :], u[1:2, :], u[2:3, :]
        vx, vy, vz = v[0:1, :], v[1:2, :], v[2:3, :]
        nx = uy * vz - uz * vy
        ny = uz * vx - ux * vz
        nz = ux * vy - uy * vx
        dd = -(nx * p1[0:1, :] + ny * p1[1:2, :] + nz * p1[2:3, :])
        nx_l.append(nx)
        ny_l.append(ny)
        nz_l.append(nz)
        dd_l.append(dd)
    NX = jnp.concatenate(nx_l, axis=0)
    NY = jnp.concatenate(ny_l, axis=0)
    NZ = jnp.concatenate(nz_l, axis=0)
    DD = jnp.concatenate(dd_l, axis=0)
    dege = (NX == 0.0) & (NY == 0.0) & (NZ == 0.0) & (DD == 0.0)
    one = jnp.ones_like(NX)
    NX = jnp.where(dege, one, NX)
    NY = jnp.where(dege, one, NY)
    NZ = jnp.where(dege, one, NZ)
    DD = jnp.where(dege, one, DD)
    norm = jnp.sqrt(NX ** 2 + NY ** 2 + NZ ** 2)
    NX, NY, NZ, DD = NX / norm, NY / norm, NZ / norm, DD / norm

    score_l = []
    for k in range(U):
        n3t = jnp.concatenate(
            [NX[k:k + 1, :], NY[k:k + 1, :], NZ[k:k + 1, :]], axis=0)
        dist = (jnp.dot(x2[k], n3t, preferred_element_type=f32)
                + DD[k:k + 1, :])
        score_l.append(jnp.sum(
            jnp.exp(-(dist * dist) / (2.0 * _INLIER_SIGMA2)),
            axis=0, keepdims=True))
    SCORE = jnp.concatenate(score_l, axis=0)

    tgt = tgt_ref[:, 0, :]
    tx, ty, tz = tgt[:, 0:1], tgt[:, 1:2], tgt[:, 2:3]
    lm = (NX - tx) ** 2 + (NY - ty) ** 2 + (NZ - tz) ** 2
    lp = (NX + tx) ** 2 + (NY + ty) ** 2 + (NZ + tz) ** 2
    LOSS = jnp.minimum(lm, lp)

    ms = jnp.max(SCORE, axis=1, keepdims=True)
    hi = jax.lax.broadcasted_iota(jnp.int32, (1, _HYPS), 1).astype(f32)
    cand = jnp.where(SCORE == ms, hi, f32(_HYPS))
    mi = jnp.min(cand, axis=1, keepdims=True)
    mh = jnp.where(hi == mi, one_f, zero_f)
    top_loss = jnp.sum(LOSS * mh, axis=1, keepdims=True)
    preds = [jnp.sum(n_ * mh, axis=1, keepdims=True)
             for n_ in (NX, NY, NZ)]

    z = _INLIER_ALPHA * SCORE
    zmax = jnp.max(z, axis=1, keepdims=True)
    e = jnp.exp(z - zmax)
    sm = e / jnp.sum(e, axis=1, keepdims=True)
    exp_loss = jnp.sum(LOSS * sm, axis=1, keepdims=True)

    zero = jnp.zeros((U, 3), f32)
    scal_ref[:, 0, :] = jnp.concatenate(
        [exp_loss, top_loss, preds[0], preds[1], preds[2], zero], axis=1)


def kernel(pts, target, W1, b1, W2, b2):
    B, _, N = pts.shape
    selc = _get_sel(B)
    xt = jnp.transpose(pts, (0, 2, 1))
    tri = np.tril(np.ones((N, N), np.float32), -1)
    grid = (B // _UNROLL,)

    def bmap3(i):
        return (i, 0, 0)

    def cmap(i):
        return (0, 0)

    scal, gpts_t = pl.pallas_call(
        _wdsac_block,
        grid=grid,
        in_specs=[
            pl.BlockSpec((_UNROLL, 3, N), bmap3),
            pl.BlockSpec((_UNROLL, N, 3), bmap3),
            pl.BlockSpec((_UNROLL, 1, 3), bmap3),
            pl.BlockSpec((3, 64), cmap),
            pl.BlockSpec((1, 64), cmap),
            pl.BlockSpec((64, 1), cmap),
            pl.BlockSpec((1, 1), cmap),
            pl.BlockSpec((_UNROLL, 1, 3 * _HYPS), bmap3),
            pl.BlockSpec((N, N), cmap),
        ],
        out_specs=[
            pl.BlockSpec((_UNROLL, 1, 8), bmap3),
            pl.BlockSpec((_UNROLL, 3, _NUM_GPTS), bmap3),
        ],
        out_shape=[
            jax.ShapeDtypeStruct((B, 1, 8), jnp.float32),
            jax.ShapeDtypeStruct((B, 3, _NUM_GPTS), jnp.float32),
        ],
    )(pts, xt, target.reshape(B, 1, 3), W1, b1.reshape(1, 64), W2,
      b2.reshape(1, 1),
      jnp.asarray(selc).reshape(B, 1, 3 * _HYPS),
      jnp.asarray(tri))

    exp_loss = scal[:, 0, 0]
    top_loss = scal[:, 0, 1]
    pred = scal[:, 0, 2:5]
    gpts = jnp.transpose(gpts_t, (0, 2, 1))
    return (exp_loss, top_loss, pred, gpts)


# speedup vs baseline: 5.2780x; 1.0007x over previous
import itertools

import jax
import jax.numpy as jnp
import numpy as np
from jax.experimental import pallas as pl

_NUM_GPTS = 128
_HYPS = 64
_INLIER_SIGMA2 = 0.01
_INLIER_ALPHA = 0.5

_SEL_CACHE = {}


def _threefry2x32(ks0, ks1, x0, x1):
    with np.errstate(over="ignore"):
        rot = ((13, 15, 26, 6), (17, 29, 16, 24))
        x0 = x0.astype(np.uint32).copy()
        x1 = x1.astype(np.uint32).copy()
        ks0 = np.uint32(ks0)
        ks1 = np.uint32(ks1)
        ks2 = np.uint32(ks0 ^ ks1 ^ np.uint32(0x1BD11BDA))
        x0 = x0 + ks0
        x1 = x1 + ks1
        inject = [(ks1, ks2, 1), (ks2, ks0, 2), (ks0, ks1, 3),
                  (ks1, ks2, 4), (ks2, ks0, 5)]
        for i in range(5):
            for d in rot[i % 2]:
                x0 = x0 + x1
                x1 = (x1 << np.uint32(d)) | (x1 >> np.uint32(32 - d))
                x1 = x1 ^ x0
            a, b, c = inject[i]
            x0 = x0 + a
            x1 = x1 + b + np.uint32(c)
        return x0, x1


def _np_randint_key42(size, maxval):
    with np.errstate(over="ignore"):
        b1, b2 = _threefry2x32(np.uint32(0), np.uint32(42),
                               np.zeros(2, np.uint32),
                               np.arange(2, dtype=np.uint32))
        k1 = (b1[0], b2[0])
        k2 = (b1[1], b2[1])
        lo = np.arange(size, dtype=np.uint32)
        hi = np.zeros(size, np.uint32)
        h1, h2 = _threefry2x32(k1[0], k1[1], hi, lo)
        higher = h1 ^ h2
        l1, l2 = _threefry2x32(k2[0], k2[1], hi, lo)
        lower = l1 ^ l2
        span = np.uint32(maxval)
        multiplier = np.uint32((2 ** 16) % maxval)
        multiplier = (multiplier * multiplier) % span
        off = (higher % span) * multiplier + (lower % span)
        off = off % span
        return off.astype(np.int32)


def _get_sel(B):
    if B not in _SEL_CACHE:
        idx_combi = np.array(
            list(itertools.combinations(range(_NUM_GPTS), 3)), dtype=np.int32)
        tmp = _np_randint_key42(_HYPS * B, idx_combi.shape[0])
        combos = idx_combi[tmp].reshape(B, _HYPS, 3)
        _SEL_CACHE[B] = np.ascontiguousarray(
            np.concatenate([combos[:, :, 0], combos[:, :, 1],
                            combos[:, :, 2]], axis=1))
    return _SEL_CACHE[B]


_UNROLL = 64


def _wdsac_block(pts_ref, xt_ref, tgt_ref, w1_ref, b1_ref, w2_ref, b2_ref,
                 sel_ref, tri_ref, scal_ref, gpts_ref):
    U = _UNROLL
    f32 = jnp.float32
    hp = jax.lax.Precision.HIGHEST
    N = xt_ref.shape[1]
    one_f = jnp.ones((), f32)
    zero_f = jnp.zeros((), f32)

    x3 = [pts_ref[k] for k in range(U)]
    x2 = [xt_ref[k] for k in range(U)]
    w1 = w1_ref[...]
    b1 = b1_ref[...]
    w2 = w2_ref[...]
    b2 = b2_ref[0, 0]
    tri = tri_ref[...]

    hs = [jnp.maximum(jnp.dot(x2[k], w1, preferred_element_type=f32) + b1,
                      0.0) for k in range(U)]
    pw = [jnp.dot(hs[k], w2, preferred_element_type=f32) + b2
          for k in range(U)]
    pr = [jnp.transpose(pw[k], (1, 0)) for k in range(U)]

    bf16 = jnp.bfloat16
    one_h = jnp.ones((), bf16)
    zero_h = jnp.zeros((), bf16)
    cmps = [jnp.where(pr[k] > pw[k], one_h,
                      jnp.where(pr[k] == pw[k], tri, zero_h))
            for k in range(U)]
    ones_col = jnp.ones((N, 1), bf16)
    ranks = [jnp.dot(cmps[k], ones_col, preferred_element_type=f32)
             for k in range(U)]

    r_iota = jax.lax.broadcasted_iota(
        jnp.int32, (N, _NUM_GPTS), 1).astype(f32)
    ohts = [jnp.where(ranks[k] == r_iota, one_f, zero_f) for k in range(U)]
    gpts_t = [jnp.dot(x3[k], ohts[k], precision=hp,
                      preferred_element_type=f32) for k in range(U)]
    for k in range(U):
        gpts_ref[k] = gpts_t[k]

    hr_iota = jax.lax.broadcasted_iota(jnp.int32, (_NUM_GPTS, 3 * _HYPS), 0)
    ohs = [jnp.where(sel_ref[k] == hr_iota, one_f, zero_f)
           for k in range(U)]
    psc = [jnp.dot(gpts_t[k], ohs[k], precision=hp,
                   preferred_element_type=f32) for k in range(U)]
    ps = [[psc[k][:, 0:_HYPS], psc[k][:, _HYPS:2 * _HYPS],
           psc[k][:, 2 * _HYPS:3 * _HYPS]] for k in range(U)]

    nx_l, ny_l, nz_l, dd_l = [], [], [], []
    for k in range(U):
        p1, p2, p3 = ps[k]
        u = p2 - p1
        v = p3 - p1
        ux, uy, uz = u[0:1, :], u[1:2, :], u[2:3, :]
        vx, vy, vz = v[0:1, :], v[1:2, :], v[2:3, :]
        nx = uy * vz - uz * vy
        ny = uz * vx - ux * vz
        nz = ux * vy - uy * vx
        dd = -(nx * p1[0:1, :] + ny * p1[1:2, :] + nz * p1[2:3, :])
        nx_l.append(nx)
        ny_l.append(ny)
        nz_l.append(nz)
        dd_l.append(dd)
    NX = jnp.concatenate(nx_l, axis=0)
    NY = jnp.concatenate(ny_l, axis=0)
    NZ = jnp.concatenate(nz_l, axis=0)
    DD = jnp.concatenate(dd_l, axis=0)
    dege = (NX == 0.0) & (NY == 0.0) & (NZ == 0.0) & (DD == 0.0)
    one = jnp.ones_like(NX)
    NX = jnp.where(dege, one, NX)
    NY = jnp.where(dege, one, NY)
    NZ = jnp.where(dege, one, NZ)
    DD = jnp.where(dege, one, DD)
    norm = jnp.sqrt(NX ** 2 + NY ** 2 + NZ ** 2)
    NX, NY, NZ, DD = NX / norm, NY / norm, NZ / norm, DD / norm

    score_l = []
    for k in range(U):
        n3t = jnp.concatenate(
            [NX[k:k + 1, :], NY[k:k + 1, :], NZ[k:k + 1, :]], axis=0)
        dist = (jnp.dot(x2[k], n3t, preferred_element_type=f32)
                + DD[k:k + 1, :])
        score_l.append(jnp.sum(
            jnp.exp(-(dist * dist) / (2.0 * _INLIER_SIGMA2)),
            axis=0, keepdims=True))
    SCORE = jnp.concatenate(score_l, axis=0)

    tgt = tgt_ref[:, 0, :]
    tx, ty, tz = tgt[:, 0:1], tgt[:, 1:2], tgt[:, 2:3]
    lm = (NX - tx) ** 2 + (NY - ty) ** 2 + (NZ - tz) ** 2
    lp = (NX + tx) ** 2 + (NY + ty) ** 2 + (NZ + tz) ** 2
    LOSS = jnp.minimum(lm, lp)

    ms = jnp.max(SCORE, axis=1, keepdims=True)
    hi = jax.lax.broadcasted_iota(jnp.int32, (1, _HYPS), 1).astype(f32)
    cand = jnp.where(SCORE == ms, hi, f32(_HYPS))
    mi = jnp.min(cand, axis=1, keepdims=True)
    mh = jnp.where(hi == mi, one_f, zero_f)
    top_loss = jnp.sum(LOSS * mh, axis=1, keepdims=True)
    preds = [jnp.sum(n_ * mh, axis=1, keepdims=True)
             for n_ in (NX, NY, NZ)]

    z = _INLIER_ALPHA * SCORE
    zmax = jnp.max(z, axis=1, keepdims=True)
    e = jnp.exp(z - zmax)
    sm = e / jnp.sum(e, axis=1, keepdims=True)
    exp_loss = jnp.sum(LOSS * sm, axis=1, keepdims=True)

    zero = jnp.zeros((U, 3), f32)
    scal_ref[:, 0, :] = jnp.concatenate(
        [exp_loss, top_loss, preds[0], preds[1], preds[2], zero], axis=1)


def kernel(pts, target, W1, b1, W2, b2):
    B, _, N = pts.shape
    selc = _get_sel(B)
    xt = jnp.transpose(pts, (0, 2, 1))
    tri = np.tril(np.ones((N, N), np.float32), -1)
    grid = (B // _UNROLL,)

    def bmap3(i):
        return (i, 0, 0)

    def cmap(i):
        return (0, 0)

    scal, gpts_t = pl.pallas_call(
        _wdsac_block,
        grid=grid,
        in_specs=[
            pl.BlockSpec((_UNROLL, 3, N), bmap3),
            pl.BlockSpec((_UNROLL, N, 3), bmap3),
            pl.BlockSpec((_UNROLL, 1, 3), bmap3),
            pl.BlockSpec((3, 64), cmap),
            pl.BlockSpec((1, 64), cmap),
            pl.BlockSpec((64, 1), cmap),
            pl.BlockSpec((1, 1), cmap),
            pl.BlockSpec((_UNROLL, 1, 3 * _HYPS), bmap3),
            pl.BlockSpec((N, N), cmap),
        ],
        out_specs=[
            pl.BlockSpec((_UNROLL, 1, 8), bmap3),
            pl.BlockSpec((_UNROLL, 3, _NUM_GPTS), bmap3),
        ],
        out_shape=[
            jax.ShapeDtypeStruct((B, 1, 8), jnp.float32),
            jax.ShapeDtypeStruct((B, 3, _NUM_GPTS), jnp.float32),
        ],
    )(pts, xt, target.reshape(B, 1, 3), W1, b1.reshape(1, 64), W2,
      b2.reshape(1, 1),
      jnp.asarray(selc).reshape(B, 1, 3 * _HYPS),
      jnp.asarray(tri, dtype=jnp.bfloat16))

    exp_loss = scal[:, 0, 0]
    top_loss = scal[:, 0, 1]
    pred = scal[:, 0, 2:5]
    gpts = jnp.transpose(gpts_t, (0, 2, 1))
    return (exp_loss, top_loss, pred, gpts)
